# baseline (speedup 1.0000x reference)
import functools

import jax
import jax.numpy as jnp
import numpy as np
from jax import lax
from jax.experimental import pallas as pl
from jax.experimental.pallas import tpu as pltpu
from jax.experimental.pallas import tpu_sc as plsc

N = 50000
E = 800000
B = 50
NUM_LAYERS = 2
NUM_CLASSES = 25

NC = 2
NS = 16
NW = NC * NS
NB = 128
BW = 391
NBW = NB * BW
BPW = NB // NW
CAP = E + 4096
PCH = 512
GCH = 128
SCH = 256
E_ALLOC = 866304

_MESH = dict(core_axis_name="c", subcore_axis_name="s")
_SC_PARAMS = pltpu.CompilerParams(needs_layout_passes=False)


def _mish(x):
    return x * jnp.tanh(jax.nn.softplus(x))


def _ln(x, g, b, eps=1e-5):
    mu = jnp.mean(x, axis=-1, keepdims=True)
    var = jnp.mean((x - mu) ** 2, axis=-1, keepdims=True)
    return (x - mu) / jnp.sqrt(var + eps) * g + b


def _conv_mask():
    M = np.zeros((9, 25, 25), np.float32)
    for di in range(3):
        for dj in range(3):
            k = di * 3 + dj
            for pi in range(5):
                for pj in range(5):
                    qi, qj = pi + di - 1, pj + dj - 1
                    if 0 <= qi < 5 and 0 <= qj < 5:
                        M[k, qi * 5 + qj, pi * 5 + pj] = 1.0
    return M


_CONV_M = _conv_mask()
_POOL_P = np.kron(np.eye(64, dtype=np.float32),
                  np.full((25, 1), 1.0 / 25, np.float32))


def _conv_as_matmul_weights(w, b, g, beta):
    co, ci = w.shape[0], w.shape[1]
    wf = (w * g[:, None, None, None]).reshape(co, ci, 9)
    bias = b * g + beta
    big = jnp.einsum('oik,kqp->iqop', wf, jnp.asarray(_CONV_M))
    return big.reshape(ci * 25, co * 25), jnp.repeat(bias, 25)


def _iota16():
    return lax.iota(jnp.int32, 16)


def _al(x, n=8):
    return pl.multiple_of(x, n)


def _vextract(vbuf, ref_idx):
    idxv = jnp.full((16,), ref_idx, jnp.int32)
    return plsc.load_gather(vbuf, [idxv])[0]


def _sc_bucket_body(src_hbm, dst_hbm, idr_hbm, srr_hbm, dsr_hbm, lens_hbm,
                    dbuf, sbuf, oid, osr, ods, idsc, srcsc, dstsc, lbuf,
                    scur, sem):
    c = lax.axis_index("c")
    s = lax.axis_index("s")
    w = s * NC + c
    CHB = 2000
    NCH = E // CHB
    lo0 = w * (BPW * BW)
    hi0 = lo0 + BPW * BW
    for j in range(BPW):
        scur[j] = 0
        scur[BPW + j] = 0

    def chunk_body(ci, _):
        pltpu.sync_copy(dst_hbm.at[pl.ds(ci * CHB, CHB)], dbuf)
        pltpu.sync_copy(src_hbm.at[pl.ds(ci * CHB, CHB)], sbuf)

        def vreg_body(i, _):
            v = dbuf[pl.ds(i * 16, 16)]
            m = (v >= lo0) & (v < hi0)
            cnt = plsc.all_reduce_population_count(m)[0]

            @pl.when(cnt > 0)
            def _extract():
                idsc[...] = (ci * CHB + i * 16) + _iota16()
                dstsc[...] = v
                srcsc[...] = sbuf[pl.ds(i * 16, 16)]

                def match_body(k, mvec):
                    fv = plsc.all_reduce_ffs(mvec)
                    idsp = plsc.load_gather(idsc, [fv])
                    ssp = plsc.load_gather(srcsc, [fv])
                    dsp = plsc.load_gather(dstsc, [fv])
                    d2 = dsp[0] - lo0
                    jj = ((d2 >= BW).astype(jnp.int32)
                          + (d2 >= 2 * BW).astype(jnp.int32)
                          + (d2 >= 3 * BW).astype(jnp.int32))
                    posj = scur[jj]
                    tgt = jnp.full((16,), jj * 4096 + posj, jnp.int32)
                    plsc.store_scatter(oid, [tgt], idsp)
                    plsc.store_scatter(osr, [tgt], ssp)
                    plsc.store_scatter(ods, [tgt], dsp)
                    scur[jj] = posj + 1
                    return mvec & (_iota16() != fv)

                lax.fori_loop(0, cnt, match_body, m)

            return 0

        lax.fori_loop(0, CHB // 16, vreg_body, 0)
        for j in range(BPW):
            @pl.when(scur[j] >= 2048)
            def _flush(j=j):
                base = scur[BPW + j]
                hb = _al((w * BPW + j) * CAP + base)
                pltpu.sync_copy(oid.at[pl.ds(j * 4096, 2048)],
                                idr_hbm.at[pl.ds(hb, 2048)])
                pltpu.sync_copy(osr.at[pl.ds(j * 4096, 2048)],
                                srr_hbm.at[pl.ds(hb, 2048)])
                pltpu.sync_copy(ods.at[pl.ds(j * 4096, 2048)],
                                dsr_hbm.at[pl.ds(hb, 2048)])
                nt = scur[j] - 2048

                def mv(k, _):
                    d0 = _al(j * 4096 + k * 16, 16)
                    d1 = _al(j * 4096 + 2048 + k * 16, 16)
                    oid[pl.ds(d0, 16)] = oid[pl.ds(d1, 16)]
                    osr[pl.ds(d0, 16)] = osr[pl.ds(d1, 16)]
                    ods[pl.ds(d0, 16)] = ods[pl.ds(d1, 16)]
                    return 0

                lax.fori_loop(0, (nt + 15) >> 4, mv, 0)
                scur[j] = nt
                scur[BPW + j] = base + 2048

        return 0

    lax.fori_loop(0, NCH, chunk_body, 0)
    for j in range(BPW):
        hb = _al((w * BPW + j) * CAP + scur[BPW + j])
        pltpu.sync_copy(oid.at[pl.ds(j * 4096, 2048)], idr_hbm.at[pl.ds(hb, 2048)])
        pltpu.sync_copy(osr.at[pl.ds(j * 4096, 2048)], srr_hbm.at[pl.ds(hb, 2048)])
        pltpu.sync_copy(ods.at[pl.ds(j * 4096, 2048)], dsr_hbm.at[pl.ds(hb, 2048)])
        lbuf[pl.ds(j * 16, 16)] = jnp.full((16,), scur[BPW + j] + scur[j],
                                           jnp.int32)
    pltpu.sync_copy(lbuf, lens_hbm.at[pl.ds(_al(w * BPW * 16), BPW * 16)])


def _sc_bucket(src, dst):
    return pl.kernel(
        _sc_bucket_body,
        out_type=[
            jax.ShapeDtypeStruct((NB * CAP,), jnp.int32),
            jax.ShapeDtypeStruct((NB * CAP,), jnp.int32),
            jax.ShapeDtypeStruct((NB * CAP,), jnp.int32),
            jax.ShapeDtypeStruct((NB * 16,), jnp.int32),
        ],
        mesh=plsc.VectorSubcoreMesh(**_MESH),
        compiler_params=_SC_PARAMS,
        scratch_types=[
            pltpu.VMEM((2000,), jnp.int32),
            pltpu.VMEM((2000,), jnp.int32),
            pltpu.VMEM((BPW * 4096,), jnp.int32),
            pltpu.VMEM((BPW * 4096,), jnp.int32),
            pltpu.VMEM((BPW * 4096,), jnp.int32),
            pltpu.VMEM((16,), jnp.int32),
            pltpu.VMEM((16,), jnp.int32),
            pltpu.VMEM((16,), jnp.int32),
            pltpu.VMEM((BPW * 16,), jnp.int32),
            pltpu.SMEM((2 * BPW,), jnp.int32),
            pltpu.SemaphoreType.DMA,
        ],
    )(src, dst)


def _sc_permute_body(idr_hbm, srr_hbm, dsr_hbm, lens_hbm, offs_hbm, e0_hbm,
                     srcp_hbm, dstp_hbm, e0p_hbm,
                     ibuf, svbuf, dvbuf, ebuf, lvbuf, ovbuf, sem):
    c = lax.axis_index("c")
    s = lax.axis_index("s")
    w = s * NC + c
    pltpu.sync_copy(lens_hbm, lvbuf)
    pltpu.sync_copy(offs_hbm, ovbuf)
    for j in range(BPW):
        b = w * BPW + j
        lo = b * BW
        n = _vextract(lvbuf, b)
        off = _vextract(ovbuf, b)
        nch = (n + PCH - 1) >> 9

        def chunk_body(ci, _):
            rb = _al(b * CAP + ci * PCH)
            pltpu.sync_copy(idr_hbm.at[pl.ds(rb, PCH)], ibuf)
            pltpu.sync_copy(srr_hbm.at[pl.ds(rb, PCH)], svbuf)
            pltpu.sync_copy(dsr_hbm.at[pl.ds(rb, PCH)], dvbuf)

            def sanitize(i, _):
                g = ci * PCH + i * 16 + _iota16()
                ok = g < n
                ibuf[pl.ds(i * 16, 16)] = jnp.where(ok, ibuf[pl.ds(i * 16, 16)], 0)
                svbuf[pl.ds(i * 16, 16)] = jnp.where(ok, svbuf[pl.ds(i * 16, 16)],
                                                     0)
                dvbuf[pl.ds(i * 16, 16)] = jnp.where(ok, dvbuf[pl.ds(i * 16, 16)],
                                                     lo + BW)
                return 0

            lax.fori_loop(0, PCH // 16, sanitize, 0)
            pltpu.async_copy(e0_hbm.at[ibuf], ebuf, sem).wait()
            ob = _al(off + ci * PCH)
            pltpu.sync_copy(svbuf, srcp_hbm.at[pl.ds(ob, PCH)])
            pltpu.sync_copy(dvbuf, dstp_hbm.at[pl.ds(ob, PCH)])
            pltpu.sync_copy(ebuf, e0p_hbm.at[pl.ds(ob, PCH)])
            return 0

        lax.fori_loop(0, nch, chunk_body, 0)


def _sc_permute(idr, srr, dsr, lens, offs, e0):
    return pl.kernel(
        _sc_permute_body,
        out_type=[
            jax.ShapeDtypeStruct((E_ALLOC,), jnp.int32),
            jax.ShapeDtypeStruct((E_ALLOC,), jnp.int32),
            jax.ShapeDtypeStruct((E_ALLOC, 128), jnp.float32),
        ],
        mesh=plsc.VectorSubcoreMesh(**_MESH),
        compiler_params=_SC_PARAMS,
        scratch_types=[
            pltpu.VMEM((PCH,), jnp.int32),
            pltpu.VMEM((PCH,), jnp.int32),
            pltpu.VMEM((PCH,), jnp.int32),
            pltpu.VMEM((PCH, 128), jnp.float32),
            pltpu.VMEM((NB,), jnp.int32),
            pltpu.VMEM((NB,), jnp.int32),
            pltpu.SemaphoreType.DMA,
        ],
    )(idr, srr, dsr, lens, offs, e0)


def _sc_gather_body(tsrc_hbm, tdst_hbm, srcp_hbm, dstp_hbm, lens_hbm, offs_hbm,
                    g_hbm, sibuf, dibuf, gbuf, g2buf, lvbuf, ovbuf, sem, sem2):
    c = lax.axis_index("c")
    s = lax.axis_index("s")
    w = s * NC + c
    pltpu.sync_copy(lens_hbm, lvbuf)
    pltpu.sync_copy(offs_hbm, ovbuf)
    for j in range(BPW):
        b = w * BPW + j
        n = _vextract(lvbuf, b)
        off = _vextract(ovbuf, b)
        npad = ((n + PCH - 1) >> 9) << 9
        nch = npad // GCH

        def chunk_body(ci, _):
            base = _al(off + ci * GCH)
            pltpu.sync_copy(srcp_hbm.at[pl.ds(base, GCH)], sibuf)
            pltpu.sync_copy(dstp_hbm.at[pl.ds(base, GCH)], dibuf)

            def clampv(i, _):
                dibuf[pl.ds(i * 16, 16)] = jnp.minimum(dibuf[pl.ds(i * 16, 16)],
                                                       N - 1)
                return 0

            lax.fori_loop(0, GCH // 16, clampv, 0)
            cp1 = pltpu.async_copy(tsrc_hbm.at[sibuf], gbuf, sem)
            cp2 = pltpu.async_copy(tdst_hbm.at[dibuf], g2buf, sem2)
            cp1.wait()
            cp2.wait()

            def addv(r, _):
                for k in range(16):
                    gbuf[r, pl.ds(k * 16, 16)] = (gbuf[r, pl.ds(k * 16, 16)]
                                                  + g2buf[r, pl.ds(k * 16, 16)])
                return 0

            lax.fori_loop(0, GCH, addv, 0)
            pltpu.sync_copy(gbuf, g_hbm.at[pl.ds(base, GCH)])
            return 0

        lax.fori_loop(0, nch, chunk_body, 0)


def _sc_gather(tsrc, tdst, srcp, dstp, lens, offs):
    return pl.kernel(
        _sc_gather_body,
        out_type=jax.ShapeDtypeStruct((E_ALLOC, 256), jnp.float32),
        mesh=plsc.VectorSubcoreMesh(**_MESH),
        compiler_params=_SC_PARAMS,
        scratch_types=[
            pltpu.VMEM((GCH,), jnp.int32),
            pltpu.VMEM((GCH,), jnp.int32),
            pltpu.VMEM((GCH, 256), jnp.float32),
            pltpu.VMEM((GCH, 256), jnp.float32),
            pltpu.VMEM((NB,), jnp.int32),
            pltpu.VMEM((NB,), jnp.int32),
            pltpu.SemaphoreType.DMA,
            pltpu.SemaphoreType.DMA,
        ],
    )(tsrc, tdst, srcp, dstp, lens, offs)


def _sc_scatter_body(m_hbm, dstp_hbm, lens_hbm, offs_hbm, agg_hbm,
                     mbuf, dbuf, aggbuf, lvbuf, ovbuf, sem):
    c = lax.axis_index("c")
    s = lax.axis_index("s")
    w = s * NC + c
    pltpu.sync_copy(lens_hbm, lvbuf)
    pltpu.sync_copy(offs_hbm, ovbuf)
    zeros = jnp.zeros((16,), jnp.float32)
    for j in range(BPW):
        b = w * BPW + j
        lo = b * BW
        n = _vextract(lvbuf, b)
        off = _vextract(ovbuf, b)
        npad = ((n + PCH - 1) >> 9) << 9
        nch = npad // SCH

        def zero_body(k, _):
            aggbuf[pl.ds(_al(k * 16, 16), 16)] = zeros
            return 0

        lax.fori_loop(0, (BW + 1) * 128 // 16, zero_body, 0)

        def chunk_body(ci, _):
            base = _al(off + ci * SCH)
            pltpu.sync_copy(m_hbm.at[pl.ds(_al(base * 128), SCH * 128)], mbuf)
            pltpu.sync_copy(dstp_hbm.at[pl.ds(base, SCH)], dbuf)

            def grp_body(g, _):
                rows = (g * 16 + _iota16()) * 128
                dv = dbuf[pl.ds(g * 16, 16)]
                doff = jnp.clip(dv - lo, 0, BW) * 128

                def col_body(cc, _):
                    for u in range(8):
                        cidx = cc * 8 + u
                        v = plsc.load_gather(mbuf, [rows + cidx])
                        plsc.addupdate_scatter(aggbuf, [doff + cidx], v)
                    return 0

                lax.fori_loop(0, 16, col_body, 0)
                return 0

            lax.fori_loop(0, SCH // 16, grp_body, 0)
            return 0

        lax.fori_loop(0, nch, chunk_body, 0)
        pltpu.sync_copy(aggbuf.at[pl.ds(0, BW * 128)],
                        agg_hbm.at[pl.ds(_al(lo * 128), BW * 128)])


def _sc_scatter(m, dstp, lens, offs):
    m_flat = m.reshape(E_ALLOC * 128)
    out = pl.kernel(
        _sc_scatter_body,
        out_type=jax.ShapeDtypeStruct((NBW * 128,), jnp.float32),
        mesh=plsc.VectorSubcoreMesh(**_MESH),
        compiler_params=_SC_PARAMS,
        scratch_types=[
            pltpu.VMEM((SCH * 128,), jnp.float32),
            pltpu.VMEM((SCH,), jnp.int32),
            pltpu.VMEM(((BW + 1) * 128,), jnp.float32),
            pltpu.VMEM((NB,), jnp.int32),
            pltpu.VMEM((NB,), jnp.int32),
            pltpu.SemaphoreType.DMA,
        ],
    )(m_flat, dstp, lens, offs)
    return out.reshape(NBW, 128)


def _node_enc_kernel(nx_ref, xg_ref, w1_ref, b1_ref, w2_ref, b2_ref, mg_ref,
                     mbe_ref, nw1_ref, nb1_ref, ng1_ref, nbe1_ref, nw2_ref,
                     nb2_ref, ng2_ref, nbe2_ref, cw1_ref, cb1_ref, cw2_ref,
                     cb2_ref, cw3_ref, cb3_ref, pp_ref, out_ref):
    x = nx_ref[...]
    hid = jnp.maximum(jnp.dot(x, w1_ref[...], preferred_element_type=jnp.float32)
                      + b1_ref[...], 0.0)
    ma = (jnp.dot(hid, w2_ref[...], preferred_element_type=jnp.float32)
          + b2_ref[...]) * mg_ref[...] + mbe_ref[...]
    h = jnp.maximum(_ln(jnp.dot(ma, nw1_ref[...],
                                preferred_element_type=jnp.float32)
                        + nb1_ref[...], ng1_ref[...], nbe1_ref[...]), 0.0)
    h = _mish(_ln(jnp.dot(h, nw2_ref[...], preferred_element_type=jnp.float32)
                  + nb2_ref[...], ng2_ref[...], nbe2_ref[...]))
    y = _mish(jnp.dot(xg_ref[...], cw1_ref[...],
                      preferred_element_type=jnp.float32) + cb1_ref[...])
    y = _mish(jnp.dot(y, cw2_ref[...], preferred_element_type=jnp.float32)
              + cb2_ref[...])
    y = _mish(jnp.dot(y, cw3_ref[...], preferred_element_type=jnp.float32)
              + cb3_ref[...])
    g = jnp.dot(y, pp_ref[...], preferred_element_type=jnp.float32)
    out_ref[...] = jnp.concatenate([h, g], axis=1)


def _node_enc(node_x, xg, p, bw1, bb1, bw2, bb2, bw3, bb3):
    blk = 400
    return pl.pallas_call(
        _node_enc_kernel,
        grid=(N // blk,),
        in_specs=[
            pl.BlockSpec((blk, 10), lambda i: (i, 0)),
            pl.BlockSpec((blk, 175), lambda i: (i, 0)),
            pl.BlockSpec((10, 256), lambda i: (0, 0)),
            pl.BlockSpec((256,), lambda i: (0,)),
            pl.BlockSpec((256, 10), lambda i: (0, 0)),
            pl.BlockSpec((10,), lambda i: (0,)),
            pl.BlockSpec((10,), lambda i: (0,)),
            pl.BlockSpec((10,), lambda i: (0,)),
            pl.BlockSpec((10, 64), lambda i: (0, 0)),
            pl.BlockSpec((64,), lambda i: (0,)),
            pl.BlockSpec((64,), lambda i: (0,)),
            pl.BlockSpec((64,), lambda i: (0,)),
            pl.BlockSpec((64, 64), lambda i: (0, 0)),
            pl.BlockSpec((64,), lambda i: (0,)),
            pl.BlockSpec((64,), lambda i: (0,)),
            pl.BlockSpec((64,), lambda i: (0,)),
            pl.BlockSpec((175, 400), lambda i: (0, 0)),
            pl.BlockSpec((400,), lambda i: (0,)),
            pl.BlockSpec((400, 800), lambda i: (0, 0)),
            pl.BlockSpec((800,), lambda i: (0,)),
            pl.BlockSpec((800, 1600), lambda i: (0, 0)),
            pl.BlockSpec((1600,), lambda i: (0,)),
            pl.BlockSpec((1600, 64), lambda i: (0, 0)),
        ],
        out_specs=pl.BlockSpec((blk, 128), lambda i: (i, 0)),
        out_shape=jax.ShapeDtypeStruct((N, 128), jnp.float32),
    )(node_x, xg, p['ma_w1'], p['ma_b1'], p['ma_w2'], p['ma_b2'], p['ma_g'],
      p['ma_be'], p['na_w1'], p['na_b1'], p['na_g1'], p['na_be1'], p['na_w2'],
      p['na_b2'], p['na_g2'], p['na_be2'], bw1, bb1, bw2, bb2, bw3, bb3,
      jnp.asarray(_POOL_P))


def _edge_enc_kernel(x_ref, w1_ref, b1_ref, g1_ref, be1_ref, w2_ref, b2_ref,
                     g2_ref, be2_ref, out_ref):
    x = x_ref[...]
    h = jnp.maximum(_ln(jnp.dot(x, w1_ref[...],
                                preferred_element_type=jnp.float32)
                        + b1_ref[...], g1_ref[...], be1_ref[...]), 0.0)
    h = _mish(_ln(jnp.dot(h, w2_ref[...], preferred_element_type=jnp.float32)
                  + b2_ref[...], g2_ref[...], be2_ref[...]))
    out_ref[...] = jnp.concatenate([h, jnp.zeros_like(h)], axis=1)


def _edge_enc(edge_x, p):
    blk = 1000
    return pl.pallas_call(
        _edge_enc_kernel,
        grid=(E // blk,),
        in_specs=[
            pl.BlockSpec((blk, 12), lambda i: (i, 0)),
            pl.BlockSpec((12, 64), lambda i: (0, 0)),
            pl.BlockSpec((64,), lambda i: (0,)),
            pl.BlockSpec((64,), lambda i: (0,)),
            pl.BlockSpec((64,), lambda i: (0,)),
            pl.BlockSpec((64, 64), lambda i: (0, 0)),
            pl.BlockSpec((64,), lambda i: (0,)),
            pl.BlockSpec((64,), lambda i: (0,)),
            pl.BlockSpec((64,), lambda i: (0,)),
        ],
        out_specs=pl.BlockSpec((blk, 128), lambda i: (i, 0)),
        out_shape=jax.ShapeDtypeStruct((E, 128), jnp.float32),
    )(edge_x, p['ea_w1'], p['ea_b1'], p['ea_g1'], p['ea_be1'], p['ea_w2'],
      p['ea_b2'], p['ea_g2'], p['ea_be2'])


def _tables_kernel(h_ref, ws_ref, bs_ref, wd_ref, ts_ref, td_ref):
    h = h_ref[...]
    ts_ref[...] = (jnp.dot(h, ws_ref[...], preferred_element_type=jnp.float32)
                   + bs_ref[...])
    td_ref[...] = jnp.dot(h, wd_ref[...], preferred_element_type=jnp.float32)


def _tables(hcur, ws, bs, wd):
    blk = 1000
    return pl.pallas_call(
        _tables_kernel,
        grid=(N // blk,),
        in_specs=[
            pl.BlockSpec((blk, 128), lambda i: (i, 0)),
            pl.BlockSpec((128, 256), lambda i: (0, 0)),
            pl.BlockSpec((256,), lambda i: (0,)),
            pl.BlockSpec((128, 256), lambda i: (0, 0)),
        ],
        out_specs=[
            pl.BlockSpec((blk, 256), lambda i: (i, 0)),
            pl.BlockSpec((blk, 256), lambda i: (i, 0)),
        ],
        out_shape=[
            jax.ShapeDtypeStruct((N, 256), jnp.float32),
            jax.ShapeDtypeStruct((N, 256), jnp.float32),
        ],
    )(hcur, ws, bs, wd)


def _mid_kernel(g_ref, e_ref, we_ref, m_ref, enew_ref):
    e = e_ref[...][:, :64]
    t = g_ref[...] + jnp.dot(e, we_ref[...], preferred_element_type=jnp.float32)
    m_ref[...] = _mish(t[:, :128])
    en = e + _mish(t[:, 128:192])
    enew_ref[...] = jnp.concatenate([en, jnp.zeros_like(en)], axis=1)


def _mid(g, ep, we):
    blk = 512
    return pl.pallas_call(
        _mid_kernel,
        grid=(E_ALLOC // blk,),
        in_specs=[
            pl.BlockSpec((blk, 256), lambda i: (i, 0)),
            pl.BlockSpec((blk, 128), lambda i: (i, 0)),
            pl.BlockSpec((64, 256), lambda i: (0, 0)),
        ],
        out_specs=[
            pl.BlockSpec((blk, 128), lambda i: (i, 0)),
            pl.BlockSpec((blk, 128), lambda i: (i, 0)),
        ],
        out_shape=[
            jax.ShapeDtypeStruct((E_ALLOC, 128), jnp.float32),
            jax.ShapeDtypeStruct((E_ALLOC, 128), jnp.float32),
        ],
    )(g, ep, we)


def _update_kernel(h_ref, agg_ref, wh_ref, wa_ref, b_ref, g_ref, be_ref,
                   out_ref):
    h = h_ref[...]
    u = (jnp.dot(h, wh_ref[...], preferred_element_type=jnp.float32)
         + jnp.dot(agg_ref[...], wa_ref[...], preferred_element_type=jnp.float32)
         + b_ref[...])
    out_ref[...] = h + _mish(_ln(u, g_ref[...], be_ref[...]))


def _update(hcur, agg, wh, wa, bb, g, be):
    blk = 1000
    return pl.pallas_call(
        _update_kernel,
        grid=(N // blk,),
        in_specs=[
            pl.BlockSpec((blk, 128), lambda i: (i, 0)),
            pl.BlockSpec((blk, 128), lambda i: (i, 0)),
            pl.BlockSpec((128, 128), lambda i: (0, 0)),
            pl.BlockSpec((128, 128), lambda i: (0, 0)),
            pl.BlockSpec((128,), lambda i: (0,)),
            pl.BlockSpec((128,), lambda i: (0,)),
            pl.BlockSpec((128,), lambda i: (0,)),
        ],
        out_specs=pl.BlockSpec((blk, 128), lambda i: (i, 0)),
        out_shape=jax.ShapeDtypeStruct((N, 128), jnp.float32),
    )(hcur, agg, wh, wa, bb, g, be)


def _pool_kernel(h_ref, out_ref):
    out_ref[...] = (jnp.sum(h_ref[...], axis=0, keepdims=True)
                    * (1.0 / (N // B)))[None]


def _pool(node_emb):
    return pl.pallas_call(
        _pool_kernel,
        grid=(B,),
        in_specs=[pl.BlockSpec((N // B, 128), lambda i: (i, 0))],
        out_specs=pl.BlockSpec((1, 1, 128), lambda i: (i, 0, 0)),
        out_shape=jax.ShapeDtypeStruct((B, 1, 128), jnp.float32),
    )(node_emb)


def _head_kernel(ne_ref, ge_ref, w1_ref, b1_ref, g_ref, be_ref, w2_ref, b2_ref,
                 out_ref):
    ne = ne_ref[...]
    ge = jnp.broadcast_to(ge_ref[0], (ne.shape[0], 128))
    u = (jnp.dot(ne, w1_ref[:128], preferred_element_type=jnp.float32)
         + jnp.dot(ge, w1_ref[128:], preferred_element_type=jnp.float32)
         + b1_ref[...])
    sx = _mish(_ln(u, g_ref[...], be_ref[...]))
    out_ref[...] = (jnp.dot(sx, w2_ref[...], preferred_element_type=jnp.float32)
                    + b2_ref[...])


def _head(node_emb, graph_emb, w1, b1, g, be, w2, b2):
    blk = N // B
    w2p = jnp.zeros((256, 128), jnp.float32).at[:, :NUM_CLASSES].set(w2)
    b2p = jnp.zeros((128,), jnp.float32).at[:NUM_CLASSES].set(b2)
    out = pl.pallas_call(
        _head_kernel,
        grid=(B,),
        in_specs=[
            pl.BlockSpec((blk, 128), lambda i: (i, 0)),
            pl.BlockSpec((1, 1, 128), lambda i: (i, 0, 0)),
            pl.BlockSpec((256, 256), lambda i: (0, 0)),
            pl.BlockSpec((256,), lambda i: (0,)),
            pl.BlockSpec((256,), lambda i: (0,)),
            pl.BlockSpec((256,), lambda i: (0,)),
            pl.BlockSpec((256, 128), lambda i: (0, 0)),
            pl.BlockSpec((128,), lambda i: (0,)),
        ],
        out_specs=pl.BlockSpec((blk, 128), lambda i: (i, 0)),
        out_shape=jax.ShapeDtypeStruct((N, 128), jnp.float32),
    )(node_emb, graph_emb, w1, b1, g, be, w2p, b2p)
    return out[:, :NUM_CLASSES]


def _pad_cols(w, total):
    return jnp.concatenate([w, jnp.zeros((w.shape[0], total - w.shape[1]),
                                         w.dtype)], axis=1)


def kernel(node_x, node_grid, edge_x, edge_index, batch_num_nodes, params):
    p = params
    bw1, bb1 = _conv_as_matmul_weights(p['c1_w'], p['c1_b'], p['bn1_g'],
                                       p['bn1_b'])
    bw2, bb2 = _conv_as_matmul_weights(p['c2_w'], p['c2_b'], p['bn2_g'],
                                       p['bn2_b'])
    bw3, bb3 = _conv_as_matmul_weights(p['c3_w'], p['c3_b'], p['bn3_g'],
                                       p['bn3_b'])
    node_feat = _node_enc(node_x, node_grid.reshape(N, 175), p,
                          bw1, bb1, bw2, bb2, bw3, bb3)
    e0 = _edge_enc(edge_x, p)
    src = edge_index[0]
    dst = edge_index[1]
    idr, srr, dsr, lens_flat = _sc_bucket(src, dst)
    lens = lens_flat.reshape(NB, 16)[:, 0]
    lens_pad = ((lens + PCH - 1) // PCH) * PCH
    offs = jnp.concatenate([jnp.zeros((1,), jnp.int32),
                            jnp.cumsum(lens_pad)]).astype(jnp.int32)[:NB]
    srcp, dstp, ep = _sc_permute(idr, srr, dsr, lens, offs, e0)
    hcur = node_feat
    for l in range(NUM_LAYERS):
        mw, mb = p['msg_w%d' % l], p['msg_b%d' % l]
        ew, eb = p['edg_w%d' % l], p['edg_b%d' % l]
        ws = _pad_cols(jnp.concatenate([mw[:128], ew[:128]], axis=1), 256)
        bs = jnp.concatenate([mb, eb, jnp.zeros((64,), jnp.float32)])
        wd = _pad_cols(jnp.concatenate([mw[128:256], ew[128:256]], axis=1), 256)
        we = _pad_cols(jnp.concatenate([mw[256:], ew[256:]], axis=1), 256)
        tsrc, tdst = _tables(hcur, ws, bs, wd)
        gbuf = _sc_gather(tsrc, tdst, srcp, dstp, lens, offs)
        m, ep = _mid(gbuf, ep, we)
        agg = _sc_scatter(m, dstp, lens, offs)
        uw, ub = p['upd_w%d' % l], p['upd_b%d' % l]
        hcur = _update(hcur, agg[:N], uw[:128], uw[128:], ub,
                       p['uln_g%d' % l], p['uln_b%d' % l])
    node_emb = hcur
    graph_emb = _pool(node_emb)
    seg = _head(node_emb, graph_emb, p['sh_w1'], p['sh_b1'], p['sh_g'],
                p['sh_be'], p['sh_w2'], p['sh_b2'])
    return seg, node_emb


# speedup vs baseline: 1.0312x; 1.0312x over previous
import functools

import jax
import jax.numpy as jnp
import numpy as np
from jax import lax
from jax.experimental import pallas as pl
from jax.experimental.pallas import tpu as pltpu
from jax.experimental.pallas import tpu_sc as plsc

N = 50000
E = 800000
B = 50
NUM_LAYERS = 2
NUM_CLASSES = 25

NC = 2
NS = 16
NW = NC * NS
NB = 128
BW = 391
NBW = NB * BW
BPW = NB // NW
CAP = E + 4096
PCH = 512
GCH = 128
SCH = 256
E_ALLOC = 866304

_MESH = dict(core_axis_name="c", subcore_axis_name="s")
_SC_PARAMS = pltpu.CompilerParams(needs_layout_passes=False)


def _mish(x):
    return x * jnp.tanh(jax.nn.softplus(x))


def _ln(x, g, b, eps=1e-5):
    mu = jnp.mean(x, axis=-1, keepdims=True)
    var = jnp.mean((x - mu) ** 2, axis=-1, keepdims=True)
    return (x - mu) / jnp.sqrt(var + eps) * g + b


def _conv_mask():
    M = np.zeros((9, 25, 25), np.float32)
    for di in range(3):
        for dj in range(3):
            k = di * 3 + dj
            for pi in range(5):
                for pj in range(5):
                    qi, qj = pi + di - 1, pj + dj - 1
                    if 0 <= qi < 5 and 0 <= qj < 5:
                        M[k, qi * 5 + qj, pi * 5 + pj] = 1.0
    return M


_CONV_M = _conv_mask()
_POOL_P = np.kron(np.eye(64, dtype=np.float32),
                  np.full((25, 1), 1.0 / 25, np.float32))


def _conv_as_matmul_weights(w, b, g, beta):
    co, ci = w.shape[0], w.shape[1]
    wf = (w * g[:, None, None, None]).reshape(co, ci, 9)
    bias = b * g + beta
    big = jnp.einsum('oik,kqp->iqop', wf, jnp.asarray(_CONV_M))
    return big.reshape(ci * 25, co * 25), jnp.repeat(bias, 25)


def _iota16():
    return lax.iota(jnp.int32, 16)


def _al(x, n=8):
    return pl.multiple_of(x, n)


def _vextract(vbuf, ref_idx):
    idxv = jnp.full((16,), ref_idx, jnp.int32)
    return plsc.load_gather(vbuf, [idxv])[0]


def _sc_bucket_body(src_hbm, dst_hbm, idr_hbm, srr_hbm, dsr_hbm, lens_hbm,
                    dbuf, sbuf, oid, osr, ods, idsc, srcsc, dstsc, lbuf,
                    scur, sem):
    c = lax.axis_index("c")
    s = lax.axis_index("s")
    w = s * NC + c
    CHB = 2000
    NCH = E // CHB
    lo0 = w * (BPW * BW)
    hi0 = lo0 + BPW * BW
    for j in range(BPW):
        scur[j] = 0
        scur[BPW + j] = 0

    def chunk_body(ci, _):
        pltpu.sync_copy(dst_hbm.at[pl.ds(ci * CHB, CHB)], dbuf)
        pltpu.sync_copy(src_hbm.at[pl.ds(ci * CHB, CHB)], sbuf)

        def vreg_body(i, _):
            v = dbuf[pl.ds(i * 16, 16)]
            m = (v >= lo0) & (v < hi0)
            cnt = plsc.all_reduce_population_count(m)[0]

            @pl.when(cnt > 0)
            def _extract():
                idsc[...] = (ci * CHB + i * 16) + _iota16()
                dstsc[...] = v
                srcsc[...] = sbuf[pl.ds(i * 16, 16)]

                def match_body(k, mvec):
                    fv = plsc.all_reduce_ffs(mvec)
                    idsp = plsc.load_gather(idsc, [fv])
                    ssp = plsc.load_gather(srcsc, [fv])
                    dsp = plsc.load_gather(dstsc, [fv])
                    d2 = dsp[0] - lo0
                    jj = ((d2 >= BW).astype(jnp.int32)
                          + (d2 >= 2 * BW).astype(jnp.int32)
                          + (d2 >= 3 * BW).astype(jnp.int32))
                    posj = scur[jj]
                    tgt = jnp.full((16,), jj * 4096 + posj, jnp.int32)
                    plsc.store_scatter(oid, [tgt], idsp)
                    plsc.store_scatter(osr, [tgt], ssp)
                    plsc.store_scatter(ods, [tgt], dsp)
                    scur[jj] = posj + 1
                    return mvec & (_iota16() != fv)

                lax.fori_loop(0, cnt, match_body, m)

            return 0

        lax.fori_loop(0, CHB // 16, vreg_body, 0)
        for j in range(BPW):
            @pl.when(scur[j] >= 2048)
            def _flush(j=j):
                base = scur[BPW + j]
                hb = _al((w * BPW + j) * CAP + base)
                pltpu.sync_copy(oid.at[pl.ds(j * 4096, 2048)],
                                idr_hbm.at[pl.ds(hb, 2048)])
                pltpu.sync_copy(osr.at[pl.ds(j * 4096, 2048)],
                                srr_hbm.at[pl.ds(hb, 2048)])
                pltpu.sync_copy(ods.at[pl.ds(j * 4096, 2048)],
                                dsr_hbm.at[pl.ds(hb, 2048)])
                nt = scur[j] - 2048

                def mv(k, _):
                    d0 = _al(j * 4096 + k * 16, 16)
                    d1 = _al(j * 4096 + 2048 + k * 16, 16)
                    oid[pl.ds(d0, 16)] = oid[pl.ds(d1, 16)]
                    osr[pl.ds(d0, 16)] = osr[pl.ds(d1, 16)]
                    ods[pl.ds(d0, 16)] = ods[pl.ds(d1, 16)]
                    return 0

                lax.fori_loop(0, (nt + 15) >> 4, mv, 0)
                scur[j] = nt
                scur[BPW + j] = base + 2048

        return 0

    lax.fori_loop(0, NCH, chunk_body, 0)
    for j in range(BPW):
        hb = _al((w * BPW + j) * CAP + scur[BPW + j])
        pltpu.sync_copy(oid.at[pl.ds(j * 4096, 2048)], idr_hbm.at[pl.ds(hb, 2048)])
        pltpu.sync_copy(osr.at[pl.ds(j * 4096, 2048)], srr_hbm.at[pl.ds(hb, 2048)])
        pltpu.sync_copy(ods.at[pl.ds(j * 4096, 2048)], dsr_hbm.at[pl.ds(hb, 2048)])
        lbuf[pl.ds(j * 16, 16)] = jnp.full((16,), scur[BPW + j] + scur[j],
                                           jnp.int32)
    pltpu.sync_copy(lbuf, lens_hbm.at[pl.ds(_al(w * BPW * 16), BPW * 16)])


def _sc_bucket(src, dst):
    return pl.kernel(
        _sc_bucket_body,
        out_type=[
            jax.ShapeDtypeStruct((NB * CAP,), jnp.int32),
            jax.ShapeDtypeStruct((NB * CAP,), jnp.int32),
            jax.ShapeDtypeStruct((NB * CAP,), jnp.int32),
            jax.ShapeDtypeStruct((NB * 16,), jnp.int32),
        ],
        mesh=plsc.VectorSubcoreMesh(**_MESH),
        compiler_params=_SC_PARAMS,
        scratch_types=[
            pltpu.VMEM((2000,), jnp.int32),
            pltpu.VMEM((2000,), jnp.int32),
            pltpu.VMEM((BPW * 4096,), jnp.int32),
            pltpu.VMEM((BPW * 4096,), jnp.int32),
            pltpu.VMEM((BPW * 4096,), jnp.int32),
            pltpu.VMEM((16,), jnp.int32),
            pltpu.VMEM((16,), jnp.int32),
            pltpu.VMEM((16,), jnp.int32),
            pltpu.VMEM((BPW * 16,), jnp.int32),
            pltpu.SMEM((2 * BPW,), jnp.int32),
            pltpu.SemaphoreType.DMA,
        ],
    )(src, dst)


def _sc_permute_body(idr_hbm, srr_hbm, dsr_hbm, lens_hbm, offs_hbm, e0_hbm,
                     srcp_hbm, dstp_hbm, e0p_hbm,
                     ibuf, svbuf, dvbuf, ebuf, lvbuf, ovbuf, sem):
    c = lax.axis_index("c")
    s = lax.axis_index("s")
    w = s * NC + c
    pltpu.sync_copy(lens_hbm, lvbuf)
    pltpu.sync_copy(offs_hbm, ovbuf)
    for j in range(BPW):
        b = w * BPW + j
        lo = b * BW
        n = _vextract(lvbuf, b)
        off = _vextract(ovbuf, b)
        nch = (n + PCH - 1) >> 9

        def chunk_body(ci, _):
            rb = _al(b * CAP + ci * PCH)
            pltpu.sync_copy(idr_hbm.at[pl.ds(rb, PCH)], ibuf)
            pltpu.sync_copy(srr_hbm.at[pl.ds(rb, PCH)], svbuf)
            pltpu.sync_copy(dsr_hbm.at[pl.ds(rb, PCH)], dvbuf)

            def sanitize(i, _):
                g = ci * PCH + i * 16 + _iota16()
                ok = g < n
                ibuf[pl.ds(i * 16, 16)] = jnp.where(ok, ibuf[pl.ds(i * 16, 16)], 0)
                svbuf[pl.ds(i * 16, 16)] = jnp.where(ok, svbuf[pl.ds(i * 16, 16)],
                                                     0)
                dvbuf[pl.ds(i * 16, 16)] = jnp.where(ok, dvbuf[pl.ds(i * 16, 16)],
                                                     lo + BW)
                return 0

            lax.fori_loop(0, PCH // 16, sanitize, 0)
            pltpu.async_copy(e0_hbm.at[ibuf], ebuf, sem).wait()
            ob = _al(off + ci * PCH)
            pltpu.sync_copy(svbuf, srcp_hbm.at[pl.ds(ob, PCH)])
            pltpu.sync_copy(dvbuf, dstp_hbm.at[pl.ds(ob, PCH)])
            pltpu.sync_copy(ebuf, e0p_hbm.at[pl.ds(ob, PCH)])
            return 0

        lax.fori_loop(0, nch, chunk_body, 0)


def _sc_permute(idr, srr, dsr, lens, offs, e0):
    return pl.kernel(
        _sc_permute_body,
        out_type=[
            jax.ShapeDtypeStruct((E_ALLOC,), jnp.int32),
            jax.ShapeDtypeStruct((E_ALLOC,), jnp.int32),
            jax.ShapeDtypeStruct((E_ALLOC, 128), jnp.float32),
        ],
        mesh=plsc.VectorSubcoreMesh(**_MESH),
        compiler_params=_SC_PARAMS,
        scratch_types=[
            pltpu.VMEM((PCH,), jnp.int32),
            pltpu.VMEM((PCH,), jnp.int32),
            pltpu.VMEM((PCH,), jnp.int32),
            pltpu.VMEM((PCH, 128), jnp.float32),
            pltpu.VMEM((NB,), jnp.int32),
            pltpu.VMEM((NB,), jnp.int32),
            pltpu.SemaphoreType.DMA,
        ],
    )(idr, srr, dsr, lens, offs, e0)


def _sc_gather_body(tsrc_hbm, tdst_hbm, srcp_hbm, dstp_hbm, lens_hbm, offs_hbm,
                    g_hbm, sibuf, dibuf, gbuf0, gbuf1, g2buf, lvbuf, ovbuf,
                    semg, sem2, semw0, semw1):
    c = lax.axis_index("c")
    s = lax.axis_index("s")
    w = s * NC + c
    pltpu.sync_copy(lens_hbm, lvbuf)
    pltpu.sync_copy(offs_hbm, ovbuf)
    gbufs = (gbuf0, gbuf1)
    semws = (semw0, semw1)
    for j in range(BPW):
        b = w * BPW + j
        n = _vextract(lvbuf, b)
        off = _vextract(ovbuf, b)
        npad = ((n + PCH - 1) >> 9) << 9
        nch = npad // GCH

        def wait_write(ci, u):
            base = _al(off + ci * GCH)
            pltpu.make_async_copy(gbufs[u], g_hbm.at[pl.ds(base, GCH)],
                                  semws[u]).wait()

        def chunk(ci, u):
            base = _al(off + ci * GCH)
            pltpu.sync_copy(srcp_hbm.at[pl.ds(base, GCH)], sibuf)
            pltpu.sync_copy(dstp_hbm.at[pl.ds(base, GCH)], dibuf)

            def clampv(i, _):
                dibuf[pl.ds(i * 16, 16)] = jnp.minimum(dibuf[pl.ds(i * 16, 16)],
                                                       N - 1)
                return 0

            lax.fori_loop(0, GCH // 16, clampv, 0)
            cp1 = pltpu.async_copy(tsrc_hbm.at[sibuf], gbufs[u], semg)
            cp2 = pltpu.async_copy(tdst_hbm.at[dibuf], g2buf, sem2)
            cp1.wait()
            cp2.wait()
            gb = gbufs[u]

            def addv(r, _):
                for k in range(16):
                    gb[r, pl.ds(k * 16, 16)] = (gb[r, pl.ds(k * 16, 16)]
                                                + g2buf[r, pl.ds(k * 16, 16)])
                return 0

            lax.fori_loop(0, GCH, addv, 0)
            pltpu.async_copy(gb, g_hbm.at[pl.ds(base, GCH)], semws[u])

        @pl.when(nch > 0)
        def _run():
            def pair_body(cj, _):
                ci0 = cj * 2

                @pl.when(cj > 0)
                def _w0():
                    wait_write(ci0 - 2, 0)

                chunk(ci0, 0)

                @pl.when(cj > 0)
                def _w1():
                    wait_write(ci0 - 1, 1)

                chunk(ci0 + 1, 1)
                return 0

            lax.fori_loop(0, nch // 2, pair_body, 0)
            wait_write(nch - 2, 0)
            wait_write(nch - 1, 1)


def _sc_gather(tsrc, tdst, srcp, dstp, lens, offs):
    return pl.kernel(
        _sc_gather_body,
        out_type=jax.ShapeDtypeStruct((E_ALLOC, 256), jnp.float32),
        mesh=plsc.VectorSubcoreMesh(**_MESH),
        compiler_params=_SC_PARAMS,
        scratch_types=[
            pltpu.VMEM((GCH,), jnp.int32),
            pltpu.VMEM((GCH,), jnp.int32),
            pltpu.VMEM((GCH, 256), jnp.float32),
            pltpu.VMEM((GCH, 256), jnp.float32),
            pltpu.VMEM((GCH, 256), jnp.float32),
            pltpu.VMEM((NB,), jnp.int32),
            pltpu.VMEM((NB,), jnp.int32),
            pltpu.SemaphoreType.DMA,
            pltpu.SemaphoreType.DMA,
            pltpu.SemaphoreType.DMA,
            pltpu.SemaphoreType.DMA,
        ],
    )(tsrc, tdst, srcp, dstp, lens, offs)


def _sc_scatter_body(m_hbm, dstp_hbm, lens_hbm, offs_hbm, agg_hbm,
                     mbuf0, mbuf1, dbuf0, dbuf1, aggbuf, lvbuf, ovbuf,
                     sem0, sem1, semd0, semd1):
    c = lax.axis_index("c")
    s = lax.axis_index("s")
    w = s * NC + c
    pltpu.sync_copy(lens_hbm, lvbuf)
    pltpu.sync_copy(offs_hbm, ovbuf)
    zeros = jnp.zeros((16,), jnp.float32)
    mbufs = (mbuf0, mbuf1)
    dbufs = (dbuf0, dbuf1)
    sems = (sem0, sem1)
    semds = (semd0, semd1)

    for j in range(BPW):
        b = w * BPW + j
        lo = b * BW
        n = _vextract(lvbuf, b)
        off = _vextract(ovbuf, b)
        npad = ((n + PCH - 1) >> 9) << 9
        nch = npad // SCH

        def zero_body(k, _):
            base = _al(k * 256, 16)
            for u in range(16):
                aggbuf[pl.ds(base + u * 16, 16)] = zeros
            return 0

        lax.fori_loop(0, (BW + 1) * 128 // 256, zero_body, 0)

        def start(ci, u):
            base = _al(off + ci * SCH)
            return (pltpu.async_copy(m_hbm.at[pl.ds(_al(base * 128), SCH * 128)],
                                     mbufs[u], sems[u]),
                    pltpu.async_copy(dstp_hbm.at[pl.ds(base, SCH)],
                                     dbufs[u], semds[u]))

        def wait(ci, u):
            base = _al(off + ci * SCH)
            pltpu.make_async_copy(m_hbm.at[pl.ds(_al(base * 128), SCH * 128)],
                                  mbufs[u], sems[u]).wait()
            pltpu.make_async_copy(dstp_hbm.at[pl.ds(base, SCH)],
                                  dbufs[u], semds[u]).wait()

        def process(u):
            mbuf = mbufs[u]
            dbuf = dbufs[u]

            def grp_body(g, _):
                rows = (g * 16 + _iota16()) * 128
                dv = dbuf[pl.ds(g * 16, 16)]
                doff = jnp.clip(dv - lo, 0, BW) * 128
                for cidx in range(128):
                    v = plsc.load_gather(mbuf, [rows + cidx])
                    plsc.addupdate_scatter(aggbuf, [doff + cidx], v)
                return 0

            lax.fori_loop(0, SCH // 16, grp_body, 0)

        @pl.when(nch > 0)
        def _run():
            start(0, 0)

            def pair_body(cj, _):
                ci0 = cj * 2

                @pl.when(ci0 + 1 < nch)
                def _s1():
                    start(ci0 + 1, 1)

                wait(ci0, 0)
                process(0)

                @pl.when(ci0 + 2 < nch)
                def _s2():
                    start(ci0 + 2, 0)

                @pl.when(ci0 + 1 < nch)
                def _p1():
                    wait(ci0 + 1, 1)
                    process(1)

                return 0

            lax.fori_loop(0, (nch + 1) // 2, pair_body, 0)

        pltpu.sync_copy(aggbuf.at[pl.ds(0, BW * 128)],
                        agg_hbm.at[pl.ds(_al(lo * 128), BW * 128)])


def _sc_scatter(m, dstp, lens, offs):
    m_flat = m.reshape(E_ALLOC * 128)
    out = pl.kernel(
        _sc_scatter_body,
        out_type=jax.ShapeDtypeStruct((NBW * 128,), jnp.float32),
        mesh=plsc.VectorSubcoreMesh(**_MESH),
        compiler_params=_SC_PARAMS,
        scratch_types=[
            pltpu.VMEM((SCH * 128,), jnp.float32),
            pltpu.VMEM((SCH * 128,), jnp.float32),
            pltpu.VMEM((SCH,), jnp.int32),
            pltpu.VMEM((SCH,), jnp.int32),
            pltpu.VMEM(((BW + 1) * 128,), jnp.float32),
            pltpu.VMEM((NB,), jnp.int32),
            pltpu.VMEM((NB,), jnp.int32),
            pltpu.SemaphoreType.DMA,
            pltpu.SemaphoreType.DMA,
            pltpu.SemaphoreType.DMA,
            pltpu.SemaphoreType.DMA,
        ],
    )(m_flat, dstp, lens, offs)
    return out.reshape(NBW, 128)


def _node_enc_kernel(nx_ref, xg_ref, w1_ref, b1_ref, w2_ref, b2_ref, mg_ref,
                     mbe_ref, nw1_ref, nb1_ref, ng1_ref, nbe1_ref, nw2_ref,
                     nb2_ref, ng2_ref, nbe2_ref, cw1_ref, cb1_ref, cw2_ref,
                     cb2_ref, cw3_ref, cb3_ref, pp_ref, out_ref):
    x = nx_ref[...]
    hid = jnp.maximum(jnp.dot(x, w1_ref[...], preferred_element_type=jnp.float32)
                      + b1_ref[...], 0.0)
    ma = (jnp.dot(hid, w2_ref[...], preferred_element_type=jnp.float32)
          + b2_ref[...]) * mg_ref[...] + mbe_ref[...]
    h = jnp.maximum(_ln(jnp.dot(ma, nw1_ref[...],
                                preferred_element_type=jnp.float32)
                        + nb1_ref[...], ng1_ref[...], nbe1_ref[...]), 0.0)
    h = _mish(_ln(jnp.dot(h, nw2_ref[...], preferred_element_type=jnp.float32)
                  + nb2_ref[...], ng2_ref[...], nbe2_ref[...]))
    y = _mish(jnp.dot(xg_ref[...], cw1_ref[...],
                      preferred_element_type=jnp.float32) + cb1_ref[...])
    y = _mish(jnp.dot(y, cw2_ref[...], preferred_element_type=jnp.float32)
              + cb2_ref[...])
    y = _mish(jnp.dot(y, cw3_ref[...], preferred_element_type=jnp.float32)
              + cb3_ref[...])
    g = jnp.dot(y, pp_ref[...], preferred_element_type=jnp.float32)
    out_ref[...] = jnp.concatenate([h, g], axis=1)


def _node_enc(node_x, xg, p, bw1, bb1, bw2, bb2, bw3, bb3):
    blk = 400
    return pl.pallas_call(
        _node_enc_kernel,
        grid=(N // blk,),
        in_specs=[
            pl.BlockSpec((blk, 10), lambda i: (i, 0)),
            pl.BlockSpec((blk, 175), lambda i: (i, 0)),
            pl.BlockSpec((10, 256), lambda i: (0, 0)),
            pl.BlockSpec((256,), lambda i: (0,)),
            pl.BlockSpec((256, 10), lambda i: (0, 0)),
            pl.BlockSpec((10,), lambda i: (0,)),
            pl.BlockSpec((10,), lambda i: (0,)),
            pl.BlockSpec((10,), lambda i: (0,)),
            pl.BlockSpec((10, 64), lambda i: (0, 0)),
            pl.BlockSpec((64,), lambda i: (0,)),
            pl.BlockSpec((64,), lambda i: (0,)),
            pl.BlockSpec((64,), lambda i: (0,)),
            pl.BlockSpec((64, 64), lambda i: (0, 0)),
            pl.BlockSpec((64,), lambda i: (0,)),
            pl.BlockSpec((64,), lambda i: (0,)),
            pl.BlockSpec((64,), lambda i: (0,)),
            pl.BlockSpec((175, 400), lambda i: (0, 0)),
            pl.BlockSpec((400,), lambda i: (0,)),
            pl.BlockSpec((400, 800), lambda i: (0, 0)),
            pl.BlockSpec((800,), lambda i: (0,)),
            pl.BlockSpec((800, 1600), lambda i: (0, 0)),
            pl.BlockSpec((1600,), lambda i: (0,)),
            pl.BlockSpec((1600, 64), lambda i: (0, 0)),
        ],
        out_specs=pl.BlockSpec((blk, 128), lambda i: (i, 0)),
        out_shape=jax.ShapeDtypeStruct((N, 128), jnp.float32),
    )(node_x, xg, p['ma_w1'], p['ma_b1'], p['ma_w2'], p['ma_b2'], p['ma_g'],
      p['ma_be'], p['na_w1'], p['na_b1'], p['na_g1'], p['na_be1'], p['na_w2'],
      p['na_b2'], p['na_g2'], p['na_be2'], bw1, bb1, bw2, bb2, bw3, bb3,
      jnp.asarray(_POOL_P))


def _edge_enc_kernel(x_ref, w1_ref, b1_ref, g1_ref, be1_ref, w2_ref, b2_ref,
                     g2_ref, be2_ref, out_ref):
    x = x_ref[...]
    h = jnp.maximum(_ln(jnp.dot(x, w1_ref[...],
                                preferred_element_type=jnp.float32)
                        + b1_ref[...], g1_ref[...], be1_ref[...]), 0.0)
    h = _mish(_ln(jnp.dot(h, w2_ref[...], preferred_element_type=jnp.float32)
                  + b2_ref[...], g2_ref[...], be2_ref[...]))
    out_ref[...] = jnp.concatenate([h, jnp.zeros_like(h)], axis=1)


def _edge_enc(edge_x, p):
    blk = 1000
    return pl.pallas_call(
        _edge_enc_kernel,
        grid=(E // blk,),
        in_specs=[
            pl.BlockSpec((blk, 12), lambda i: (i, 0)),
            pl.BlockSpec((12, 64), lambda i: (0, 0)),
            pl.BlockSpec((64,), lambda i: (0,)),
            pl.BlockSpec((64,), lambda i: (0,)),
            pl.BlockSpec((64,), lambda i: (0,)),
            pl.BlockSpec((64, 64), lambda i: (0, 0)),
            pl.BlockSpec((64,), lambda i: (0,)),
            pl.BlockSpec((64,), lambda i: (0,)),
            pl.BlockSpec((64,), lambda i: (0,)),
        ],
        out_specs=pl.BlockSpec((blk, 128), lambda i: (i, 0)),
        out_shape=jax.ShapeDtypeStruct((E, 128), jnp.float32),
    )(edge_x, p['ea_w1'], p['ea_b1'], p['ea_g1'], p['ea_be1'], p['ea_w2'],
      p['ea_b2'], p['ea_g2'], p['ea_be2'])


def _tables_kernel(h_ref, ws_ref, bs_ref, wd_ref, ts_ref, td_ref):
    h = h_ref[...]
    ts_ref[...] = (jnp.dot(h, ws_ref[...], preferred_element_type=jnp.float32)
                   + bs_ref[...])
    td_ref[...] = jnp.dot(h, wd_ref[...], preferred_element_type=jnp.float32)


def _tables(hcur, ws, bs, wd):
    blk = 1000
    return pl.pallas_call(
        _tables_kernel,
        grid=(N // blk,),
        in_specs=[
            pl.BlockSpec((blk, 128), lambda i: (i, 0)),
            pl.BlockSpec((128, 256), lambda i: (0, 0)),
            pl.BlockSpec((256,), lambda i: (0,)),
            pl.BlockSpec((128, 256), lambda i: (0, 0)),
        ],
        out_specs=[
            pl.BlockSpec((blk, 256), lambda i: (i, 0)),
            pl.BlockSpec((blk, 256), lambda i: (i, 0)),
        ],
        out_shape=[
            jax.ShapeDtypeStruct((N, 256), jnp.float32),
            jax.ShapeDtypeStruct((N, 256), jnp.float32),
        ],
    )(hcur, ws, bs, wd)


def _mid_kernel(g_ref, e_ref, we_ref, m_ref, enew_ref):
    e = e_ref[...][:, :64]
    t = g_ref[...] + jnp.dot(e, we_ref[...], preferred_element_type=jnp.float32)
    m_ref[...] = _mish(t[:, :128])
    en = e + _mish(t[:, 128:192])
    enew_ref[...] = jnp.concatenate([en, jnp.zeros_like(en)], axis=1)


def _mid(g, ep, we):
    blk = 512
    return pl.pallas_call(
        _mid_kernel,
        grid=(E_ALLOC // blk,),
        in_specs=[
            pl.BlockSpec((blk, 256), lambda i: (i, 0)),
            pl.BlockSpec((blk, 128), lambda i: (i, 0)),
            pl.BlockSpec((64, 256), lambda i: (0, 0)),
        ],
        out_specs=[
            pl.BlockSpec((blk, 128), lambda i: (i, 0)),
            pl.BlockSpec((blk, 128), lambda i: (i, 0)),
        ],
        out_shape=[
            jax.ShapeDtypeStruct((E_ALLOC, 128), jnp.float32),
            jax.ShapeDtypeStruct((E_ALLOC, 128), jnp.float32),
        ],
    )(g, ep, we)


def _update_kernel(h_ref, agg_ref, wh_ref, wa_ref, b_ref, g_ref, be_ref,
                   out_ref):
    h = h_ref[...]
    u = (jnp.dot(h, wh_ref[...], preferred_element_type=jnp.float32)
         + jnp.dot(agg_ref[...], wa_ref[...], preferred_element_type=jnp.float32)
         + b_ref[...])
    out_ref[...] = h + _mish(_ln(u, g_ref[...], be_ref[...]))


def _update(hcur, agg, wh, wa, bb, g, be):
    blk = 1000
    return pl.pallas_call(
        _update_kernel,
        grid=(N // blk,),
        in_specs=[
            pl.BlockSpec((blk, 128), lambda i: (i, 0)),
            pl.BlockSpec((blk, 128), lambda i: (i, 0)),
            pl.BlockSpec((128, 128), lambda i: (0, 0)),
            pl.BlockSpec((128, 128), lambda i: (0, 0)),
            pl.BlockSpec((128,), lambda i: (0,)),
            pl.BlockSpec((128,), lambda i: (0,)),
            pl.BlockSpec((128,), lambda i: (0,)),
        ],
        out_specs=pl.BlockSpec((blk, 128), lambda i: (i, 0)),
        out_shape=jax.ShapeDtypeStruct((N, 128), jnp.float32),
    )(hcur, agg, wh, wa, bb, g, be)


def _pool_kernel(h_ref, out_ref):
    out_ref[...] = (jnp.sum(h_ref[...], axis=0, keepdims=True)
                    * (1.0 / (N // B)))[None]


def _pool(node_emb):
    return pl.pallas_call(
        _pool_kernel,
        grid=(B,),
        in_specs=[pl.BlockSpec((N // B, 128), lambda i: (i, 0))],
        out_specs=pl.BlockSpec((1, 1, 128), lambda i: (i, 0, 0)),
        out_shape=jax.ShapeDtypeStruct((B, 1, 128), jnp.float32),
    )(node_emb)


def _head_kernel(ne_ref, ge_ref, w1_ref, b1_ref, g_ref, be_ref, w2_ref, b2_ref,
                 out_ref):
    ne = ne_ref[...]
    ge = jnp.broadcast_to(ge_ref[0], (ne.shape[0], 128))
    u = (jnp.dot(ne, w1_ref[:128], preferred_element_type=jnp.float32)
         + jnp.dot(ge, w1_ref[128:], preferred_element_type=jnp.float32)
         + b1_ref[...])
    sx = _mish(_ln(u, g_ref[...], be_ref[...]))
    out_ref[...] = (jnp.dot(sx, w2_ref[...], preferred_element_type=jnp.float32)
                    + b2_ref[...])


def _head(node_emb, graph_emb, w1, b1, g, be, w2, b2):
    blk = N // B
    w2p = jnp.zeros((256, 128), jnp.float32).at[:, :NUM_CLASSES].set(w2)
    b2p = jnp.zeros((128,), jnp.float32).at[:NUM_CLASSES].set(b2)
    out = pl.pallas_call(
        _head_kernel,
        grid=(B,),
        in_specs=[
            pl.BlockSpec((blk, 128), lambda i: (i, 0)),
            pl.BlockSpec((1, 1, 128), lambda i: (i, 0, 0)),
            pl.BlockSpec((256, 256), lambda i: (0, 0)),
            pl.BlockSpec((256,), lambda i: (0,)),
            pl.BlockSpec((256,), lambda i: (0,)),
            pl.BlockSpec((256,), lambda i: (0,)),
            pl.BlockSpec((256, 128), lambda i: (0, 0)),
            pl.BlockSpec((128,), lambda i: (0,)),
        ],
        out_specs=pl.BlockSpec((blk, 128), lambda i: (i, 0)),
        out_shape=jax.ShapeDtypeStruct((N, 128), jnp.float32),
    )(node_emb, graph_emb, w1, b1, g, be, w2p, b2p)
    return out[:, :NUM_CLASSES]


def _pad_cols(w, total):
    return jnp.concatenate([w, jnp.zeros((w.shape[0], total - w.shape[1]),
                                         w.dtype)], axis=1)


def kernel(node_x, node_grid, edge_x, edge_index, batch_num_nodes, params):
    p = params
    bw1, bb1 = _conv_as_matmul_weights(p['c1_w'], p['c1_b'], p['bn1_g'],
                                       p['bn1_b'])
    bw2, bb2 = _conv_as_matmul_weights(p['c2_w'], p['c2_b'], p['bn2_g'],
                                       p['bn2_b'])
    bw3, bb3 = _conv_as_matmul_weights(p['c3_w'], p['c3_b'], p['bn3_g'],
                                       p['bn3_b'])
    node_feat = _node_enc(node_x, node_grid.reshape(N, 175), p,
                          bw1, bb1, bw2, bb2, bw3, bb3)
    e0 = _edge_enc(edge_x, p)
    src = edge_index[0]
    dst = edge_index[1]
    idr, srr, dsr, lens_flat = _sc_bucket(src, dst)
    lens = lens_flat.reshape(NB, 16)[:, 0]
    lens_pad = ((lens + PCH - 1) // PCH) * PCH
    offs = jnp.concatenate([jnp.zeros((1,), jnp.int32),
                            jnp.cumsum(lens_pad)]).astype(jnp.int32)[:NB]
    srcp, dstp, ep = _sc_permute(idr, srr, dsr, lens, offs, e0)
    hcur = node_feat
    for l in range(NUM_LAYERS):
        mw, mb = p['msg_w%d' % l], p['msg_b%d' % l]
        ew, eb = p['edg_w%d' % l], p['edg_b%d' % l]
        ws = _pad_cols(jnp.concatenate([mw[:128], ew[:128]], axis=1), 256)
        bs = jnp.concatenate([mb, eb, jnp.zeros((64,), jnp.float32)])
        wd = _pad_cols(jnp.concatenate([mw[128:256], ew[128:256]], axis=1), 256)
        we = _pad_cols(jnp.concatenate([mw[256:], ew[256:]], axis=1), 256)
        tsrc, tdst = _tables(hcur, ws, bs, wd)
        gbuf = _sc_gather(tsrc, tdst, srcp, dstp, lens, offs)
        m, ep = _mid(gbuf, ep, we)
        agg = _sc_scatter(m, dstp, lens, offs)
        uw, ub = p['upd_w%d' % l], p['upd_b%d' % l]
        hcur = _update(hcur, agg[:N], uw[:128], uw[128:], ub,
                       p['uln_g%d' % l], p['uln_b%d' % l])
    node_emb = hcur
    graph_emb = _pool(node_emb)
    seg = _head(node_emb, graph_emb, p['sh_w1'], p['sh_b1'], p['sh_g'],
                p['sh_be'], p['sh_w2'], p['sh_b2'])
    return seg, node_emb


# speedup vs baseline: 1.1649x; 1.1297x over previous
import functools

import jax
import jax.numpy as jnp
import numpy as np
from jax import lax
from jax.experimental import pallas as pl
from jax.experimental.pallas import tpu as pltpu
from jax.experimental.pallas import tpu_sc as plsc

N = 50000
E = 800000
B = 50
NUM_LAYERS = 2
NUM_CLASSES = 25

NC = 2
NS = 16
NW = NC * NS
NB = 128
BW = 391
NBW = NB * BW
BPW = NB // NW
CAP = E + 4096
PCH = 512
GCH = 256
SCH = 256
E_ALLOC = 866304

_MESH = dict(core_axis_name="c", subcore_axis_name="s")
_SC_PARAMS = pltpu.CompilerParams(needs_layout_passes=False)


def _mish(x):
    return x * jnp.tanh(jax.nn.softplus(x))


def _ln(x, g, b, eps=1e-5):
    mu = jnp.mean(x, axis=-1, keepdims=True)
    var = jnp.mean((x - mu) ** 2, axis=-1, keepdims=True)
    return (x - mu) / jnp.sqrt(var + eps) * g + b


def _conv_mask():
    M = np.zeros((9, 25, 25), np.float32)
    for di in range(3):
        for dj in range(3):
            k = di * 3 + dj
            for pi in range(5):
                for pj in range(5):
                    qi, qj = pi + di - 1, pj + dj - 1
                    if 0 <= qi < 5 and 0 <= qj < 5:
                        M[k, qi * 5 + qj, pi * 5 + pj] = 1.0
    return M


_CONV_M = _conv_mask()
_POOL_P = np.kron(np.eye(64, dtype=np.float32),
                  np.full((25, 1), 1.0 / 25, np.float32))


def _conv_as_matmul_weights(w, b, g, beta):
    co, ci = w.shape[0], w.shape[1]
    wf = (w * g[:, None, None, None]).reshape(co, ci, 9)
    bias = b * g + beta
    big = jnp.einsum('oik,kqp->iqop', wf, jnp.asarray(_CONV_M))
    return big.reshape(ci * 25, co * 25), jnp.repeat(bias, 25)


def _iota16():
    return lax.iota(jnp.int32, 16)


def _al(x, n=8):
    return pl.multiple_of(x, n)


def _vextract(vbuf, ref_idx):
    idxv = jnp.full((16,), ref_idx, jnp.int32)
    return plsc.load_gather(vbuf, [idxv])[0]


def _sc_bucket_body(src_hbm, dst_hbm, idr_hbm, srr_hbm, dsr_hbm, lens_hbm,
                    dbuf, sbuf, oid, osr, ods, idsc, srcsc, dstsc, lbuf,
                    scur, sem):
    c = lax.axis_index("c")
    s = lax.axis_index("s")
    w = s * NC + c
    CHB = 2000
    NCH = E // CHB
    lo0 = w * (BPW * BW)
    hi0 = lo0 + BPW * BW
    for j in range(BPW):
        scur[j] = 0
        scur[BPW + j] = 0

    def chunk_body(ci, _):
        pltpu.sync_copy(dst_hbm.at[pl.ds(ci * CHB, CHB)], dbuf)
        pltpu.sync_copy(src_hbm.at[pl.ds(ci * CHB, CHB)], sbuf)

        def vreg_body(i, _):
            v = dbuf[pl.ds(i * 16, 16)]
            m = (v >= lo0) & (v < hi0)
            cnt = plsc.all_reduce_population_count(m)[0]

            @pl.when(cnt > 0)
            def _extract():
                idsc[...] = (ci * CHB + i * 16) + _iota16()
                dstsc[...] = v
                srcsc[...] = sbuf[pl.ds(i * 16, 16)]

                def match_body(k, mvec):
                    fv = plsc.all_reduce_ffs(mvec)
                    idsp = plsc.load_gather(idsc, [fv])
                    ssp = plsc.load_gather(srcsc, [fv])
                    dsp = plsc.load_gather(dstsc, [fv])
                    d2 = dsp[0] - lo0
                    jj = ((d2 >= BW).astype(jnp.int32)
                          + (d2 >= 2 * BW).astype(jnp.int32)
                          + (d2 >= 3 * BW).astype(jnp.int32))
                    posj = scur[jj]
                    tgt = jnp.full((16,), jj * 4096 + posj, jnp.int32)
                    plsc.store_scatter(oid, [tgt], idsp)
                    plsc.store_scatter(osr, [tgt], ssp)
                    plsc.store_scatter(ods, [tgt], dsp)
                    scur[jj] = posj + 1
                    return mvec & (_iota16() != fv)

                lax.fori_loop(0, cnt, match_body, m)

            return 0

        lax.fori_loop(0, CHB // 16, vreg_body, 0)
        for j in range(BPW):
            @pl.when(scur[j] >= 2048)
            def _flush(j=j):
                base = scur[BPW + j]
                hb = _al((w * BPW + j) * CAP + base)
                pltpu.sync_copy(oid.at[pl.ds(j * 4096, 2048)],
                                idr_hbm.at[pl.ds(hb, 2048)])
                pltpu.sync_copy(osr.at[pl.ds(j * 4096, 2048)],
                                srr_hbm.at[pl.ds(hb, 2048)])
                pltpu.sync_copy(ods.at[pl.ds(j * 4096, 2048)],
                                dsr_hbm.at[pl.ds(hb, 2048)])
                nt = scur[j] - 2048

                def mv(k, _):
                    d0 = _al(j * 4096 + k * 16, 16)
                    d1 = _al(j * 4096 + 2048 + k * 16, 16)
                    oid[pl.ds(d0, 16)] = oid[pl.ds(d1, 16)]
                    osr[pl.ds(d0, 16)] = osr[pl.ds(d1, 16)]
                    ods[pl.ds(d0, 16)] = ods[pl.ds(d1, 16)]
                    return 0

                lax.fori_loop(0, (nt + 15) >> 4, mv, 0)
                scur[j] = nt
                scur[BPW + j] = base + 2048

        return 0

    lax.fori_loop(0, NCH, chunk_body, 0)
    for j in range(BPW):
        hb = _al((w * BPW + j) * CAP + scur[BPW + j])
        pltpu.sync_copy(oid.at[pl.ds(j * 4096, 2048)], idr_hbm.at[pl.ds(hb, 2048)])
        pltpu.sync_copy(osr.at[pl.ds(j * 4096, 2048)], srr_hbm.at[pl.ds(hb, 2048)])
        pltpu.sync_copy(ods.at[pl.ds(j * 4096, 2048)], dsr_hbm.at[pl.ds(hb, 2048)])
        lbuf[pl.ds(j * 16, 16)] = jnp.full((16,), scur[BPW + j] + scur[j],
                                           jnp.int32)
    pltpu.sync_copy(lbuf, lens_hbm.at[pl.ds(_al(w * BPW * 16), BPW * 16)])


def _sc_bucket(src, dst):
    return pl.kernel(
        _sc_bucket_body,
        out_type=[
            jax.ShapeDtypeStruct((NB * CAP,), jnp.int32),
            jax.ShapeDtypeStruct((NB * CAP,), jnp.int32),
            jax.ShapeDtypeStruct((NB * CAP,), jnp.int32),
            jax.ShapeDtypeStruct((NB * 16,), jnp.int32),
        ],
        mesh=plsc.VectorSubcoreMesh(**_MESH),
        compiler_params=_SC_PARAMS,
        scratch_types=[
            pltpu.VMEM((2000,), jnp.int32),
            pltpu.VMEM((2000,), jnp.int32),
            pltpu.VMEM((BPW * 4096,), jnp.int32),
            pltpu.VMEM((BPW * 4096,), jnp.int32),
            pltpu.VMEM((BPW * 4096,), jnp.int32),
            pltpu.VMEM((16,), jnp.int32),
            pltpu.VMEM((16,), jnp.int32),
            pltpu.VMEM((16,), jnp.int32),
            pltpu.VMEM((BPW * 16,), jnp.int32),
            pltpu.SMEM((2 * BPW,), jnp.int32),
            pltpu.SemaphoreType.DMA,
        ],
    )(src, dst)


def _sc_permute_body(idr_hbm, srr_hbm, dsr_hbm, lens_hbm, offs_hbm, e0_hbm,
                     srcp_hbm, dstp_hbm, e0p_hbm,
                     ibuf, svbuf, dvbuf, ebuf, lvbuf, ovbuf, sem):
    c = lax.axis_index("c")
    s = lax.axis_index("s")
    w = s * NC + c
    pltpu.sync_copy(lens_hbm, lvbuf)
    pltpu.sync_copy(offs_hbm, ovbuf)
    for j in range(BPW):
        b = w * BPW + j
        lo = b * BW
        n = _vextract(lvbuf, b)
        off = _vextract(ovbuf, b)
        nch = (n + PCH - 1) >> 9

        def chunk_body(ci, _):
            rb = _al(b * CAP + ci * PCH)
            pltpu.sync_copy(idr_hbm.at[pl.ds(rb, PCH)], ibuf)
            pltpu.sync_copy(srr_hbm.at[pl.ds(rb, PCH)], svbuf)
            pltpu.sync_copy(dsr_hbm.at[pl.ds(rb, PCH)], dvbuf)

            def sanitize(i, _):
                g = ci * PCH + i * 16 + _iota16()
                ok = g < n
                ibuf[pl.ds(i * 16, 16)] = jnp.where(ok, ibuf[pl.ds(i * 16, 16)], 0)
                svbuf[pl.ds(i * 16, 16)] = jnp.where(ok, svbuf[pl.ds(i * 16, 16)],
                                                     0)
                dvbuf[pl.ds(i * 16, 16)] = jnp.where(ok, dvbuf[pl.ds(i * 16, 16)],
                                                     lo + BW)
                return 0

            lax.fori_loop(0, PCH // 16, sanitize, 0)
            pltpu.async_copy(e0_hbm.at[ibuf], ebuf, sem).wait()
            ob = _al(off + ci * PCH)
            pltpu.sync_copy(svbuf, srcp_hbm.at[pl.ds(ob, PCH)])
            pltpu.sync_copy(dvbuf, dstp_hbm.at[pl.ds(ob, PCH)])
            pltpu.sync_copy(ebuf, e0p_hbm.at[pl.ds(ob, PCH)])
            return 0

        lax.fori_loop(0, nch, chunk_body, 0)


def _sc_permute(idr, srr, dsr, lens, offs, e0):
    return pl.kernel(
        _sc_permute_body,
        out_type=[
            jax.ShapeDtypeStruct((E_ALLOC,), jnp.int32),
            jax.ShapeDtypeStruct((E_ALLOC,), jnp.int32),
            jax.ShapeDtypeStruct((E_ALLOC, 128), jnp.float32),
        ],
        mesh=plsc.VectorSubcoreMesh(**_MESH),
        compiler_params=_SC_PARAMS,
        scratch_types=[
            pltpu.VMEM((PCH,), jnp.int32),
            pltpu.VMEM((PCH,), jnp.int32),
            pltpu.VMEM((PCH,), jnp.int32),
            pltpu.VMEM((PCH, 128), jnp.float32),
            pltpu.VMEM((NB,), jnp.int32),
            pltpu.VMEM((NB,), jnp.int32),
            pltpu.SemaphoreType.DMA,
        ],
    )(idr, srr, dsr, lens, offs, e0)


def _sc_gather_body(h_hbm, srcp_hbm, dstp_hbm, lens_hbm, offs_hbm,
                    g1_hbm, g2_hbm, sibuf, dibuf, g1buf, g2buf, lvbuf, ovbuf,
                    semg, sem2, semw0, semw1):
    c = lax.axis_index("c")
    s = lax.axis_index("s")
    w = s * NC + c
    pltpu.sync_copy(lens_hbm, lvbuf)
    pltpu.sync_copy(offs_hbm, ovbuf)
    for j in range(BPW):
        b = w * BPW + j
        n = _vextract(lvbuf, b)
        off = _vextract(ovbuf, b)
        npad = ((n + PCH - 1) >> 9) << 9
        nch = npad // GCH

        def wait_writes(ci):
            base = _al(off + ci * GCH)
            pltpu.make_async_copy(g1buf, g1_hbm.at[pl.ds(base, GCH)],
                                  semw0).wait()
            pltpu.make_async_copy(g2buf, g2_hbm.at[pl.ds(base, GCH)],
                                  semw1).wait()

        def chunk(ci, _):
            base = _al(off + ci * GCH)
            pltpu.sync_copy(srcp_hbm.at[pl.ds(base, GCH)], sibuf)
            pltpu.sync_copy(dstp_hbm.at[pl.ds(base, GCH)], dibuf)

            def clampv(i, _):
                dibuf[pl.ds(i * 16, 16)] = jnp.minimum(dibuf[pl.ds(i * 16, 16)],
                                                       N - 1)
                return 0

            lax.fori_loop(0, GCH // 16, clampv, 0)
            cp1 = pltpu.async_copy(h_hbm.at[sibuf], g1buf, semg)
            cp2 = pltpu.async_copy(h_hbm.at[dibuf], g2buf, sem2)
            cp1.wait()
            cp2.wait()
            pltpu.async_copy(g1buf, g1_hbm.at[pl.ds(base, GCH)], semw0)
            pltpu.async_copy(g2buf, g2_hbm.at[pl.ds(base, GCH)], semw1)
            wait_writes(ci)
            return 0

        lax.fori_loop(0, nch, chunk, 0)


def _sc_gather(hcur, srcp, dstp, lens, offs):
    return pl.kernel(
        _sc_gather_body,
        out_type=[
            jax.ShapeDtypeStruct((E_ALLOC, 128), jnp.float32),
            jax.ShapeDtypeStruct((E_ALLOC, 128), jnp.float32),
        ],
        mesh=plsc.VectorSubcoreMesh(**_MESH),
        compiler_params=_SC_PARAMS,
        scratch_types=[
            pltpu.VMEM((GCH,), jnp.int32),
            pltpu.VMEM((GCH,), jnp.int32),
            pltpu.VMEM((GCH, 128), jnp.float32),
            pltpu.VMEM((GCH, 128), jnp.float32),
            pltpu.VMEM((NB,), jnp.int32),
            pltpu.VMEM((NB,), jnp.int32),
            pltpu.SemaphoreType.DMA,
            pltpu.SemaphoreType.DMA,
            pltpu.SemaphoreType.DMA,
            pltpu.SemaphoreType.DMA,
        ],
    )(hcur, srcp, dstp, lens, offs)


def _sc_scatter_body(m_hbm, dstp_hbm, lens_hbm, offs_hbm, agg_hbm,
                     mbuf0, mbuf1, dbuf0, dbuf1, aggbuf, lvbuf, ovbuf,
                     sem0, sem1, semd0, semd1):
    c = lax.axis_index("c")
    s = lax.axis_index("s")
    w = s * NC + c
    pltpu.sync_copy(lens_hbm, lvbuf)
    pltpu.sync_copy(offs_hbm, ovbuf)
    zeros = jnp.zeros((16,), jnp.float32)
    mbufs = (mbuf0, mbuf1)
    dbufs = (dbuf0, dbuf1)
    sems = (sem0, sem1)
    semds = (semd0, semd1)

    for j in range(BPW):
        b = w * BPW + j
        lo = b * BW
        n = _vextract(lvbuf, b)
        off = _vextract(ovbuf, b)
        npad = ((n + PCH - 1) >> 9) << 9
        nch = npad // SCH

        def zero_body(k, _):
            base = _al(k * 256, 16)
            for u in range(16):
                aggbuf[pl.ds(base + u * 16, 16)] = zeros
            return 0

        lax.fori_loop(0, (BW + 1) * 128 // 256, zero_body, 0)

        def start(ci, u):
            base = _al(off + ci * SCH)
            return (pltpu.async_copy(m_hbm.at[pl.ds(_al(base * 128), SCH * 128)],
                                     mbufs[u], sems[u]),
                    pltpu.async_copy(dstp_hbm.at[pl.ds(base, SCH)],
                                     dbufs[u], semds[u]))

        def wait(ci, u):
            base = _al(off + ci * SCH)
            pltpu.make_async_copy(m_hbm.at[pl.ds(_al(base * 128), SCH * 128)],
                                  mbufs[u], sems[u]).wait()
            pltpu.make_async_copy(dstp_hbm.at[pl.ds(base, SCH)],
                                  dbufs[u], semds[u]).wait()

        def process(u):
            mbuf = mbufs[u]
            dbuf = dbufs[u]

            def grp_body(g, _):
                rows = (g * 16 + _iota16()) * 128
                dv = dbuf[pl.ds(g * 16, 16)]
                doff = jnp.clip(dv - lo, 0, BW) * 128
                for cc in range(16):
                    vs = [plsc.load_gather(mbuf, [rows + cc * 8 + u])
                          for u in range(8)]
                    for u in range(8):
                        plsc.addupdate_scatter(aggbuf, [doff + cc * 8 + u],
                                               vs[u])
                return 0

            lax.fori_loop(0, SCH // 16, grp_body, 0)

        @pl.when(nch > 0)
        def _run():
            start(0, 0)

            def pair_body(cj, _):
                ci0 = cj * 2

                @pl.when(ci0 + 1 < nch)
                def _s1():
                    start(ci0 + 1, 1)

                wait(ci0, 0)
                process(0)

                @pl.when(ci0 + 2 < nch)
                def _s2():
                    start(ci0 + 2, 0)

                @pl.when(ci0 + 1 < nch)
                def _p1():
                    wait(ci0 + 1, 1)
                    process(1)

                return 0

            lax.fori_loop(0, (nch + 1) // 2, pair_body, 0)

        pltpu.sync_copy(aggbuf.at[pl.ds(0, BW * 128)],
                        agg_hbm.at[pl.ds(_al(lo * 128), BW * 128)])


def _sc_scatter(m, dstp, lens, offs):
    m_flat = m.reshape(E_ALLOC * 128)
    out = pl.kernel(
        _sc_scatter_body,
        out_type=jax.ShapeDtypeStruct((NBW * 128,), jnp.float32),
        mesh=plsc.VectorSubcoreMesh(**_MESH),
        compiler_params=_SC_PARAMS,
        scratch_types=[
            pltpu.VMEM((SCH * 128,), jnp.float32),
            pltpu.VMEM((SCH * 128,), jnp.float32),
            pltpu.VMEM((SCH,), jnp.int32),
            pltpu.VMEM((SCH,), jnp.int32),
            pltpu.VMEM(((BW + 1) * 128,), jnp.float32),
            pltpu.VMEM((NB,), jnp.int32),
            pltpu.VMEM((NB,), jnp.int32),
            pltpu.SemaphoreType.DMA,
            pltpu.SemaphoreType.DMA,
            pltpu.SemaphoreType.DMA,
            pltpu.SemaphoreType.DMA,
        ],
    )(m_flat, dstp, lens, offs)
    return out.reshape(NBW, 128)


def _node_enc_kernel(nx_ref, xg_ref, w1_ref, b1_ref, w2_ref, b2_ref, mg_ref,
                     mbe_ref, nw1_ref, nb1_ref, ng1_ref, nbe1_ref, nw2_ref,
                     nb2_ref, ng2_ref, nbe2_ref, cw1_ref, cb1_ref, cw2_ref,
                     cb2_ref, cw3_ref, cb3_ref, pp_ref, out_ref):
    x = nx_ref[...]
    hid = jnp.maximum(jnp.dot(x, w1_ref[...], preferred_element_type=jnp.float32)
                      + b1_ref[...], 0.0)
    ma = (jnp.dot(hid, w2_ref[...], preferred_element_type=jnp.float32)
          + b2_ref[...]) * mg_ref[...] + mbe_ref[...]
    h = jnp.maximum(_ln(jnp.dot(ma, nw1_ref[...],
                                preferred_element_type=jnp.float32)
                        + nb1_ref[...], ng1_ref[...], nbe1_ref[...]), 0.0)
    h = _mish(_ln(jnp.dot(h, nw2_ref[...], preferred_element_type=jnp.float32)
                  + nb2_ref[...], ng2_ref[...], nbe2_ref[...]))
    y = _mish(jnp.dot(xg_ref[...], cw1_ref[...],
                      preferred_element_type=jnp.float32) + cb1_ref[...])
    y = _mish(jnp.dot(y, cw2_ref[...], preferred_element_type=jnp.float32)
              + cb2_ref[...])
    y = _mish(jnp.dot(y, cw3_ref[...], preferred_element_type=jnp.float32)
              + cb3_ref[...])
    g = jnp.dot(y, pp_ref[...], preferred_element_type=jnp.float32)
    out_ref[...] = jnp.concatenate([h, g], axis=1)


def _node_enc(node_x, xg, p, bw1, bb1, bw2, bb2, bw3, bb3):
    blk = 400
    return pl.pallas_call(
        _node_enc_kernel,
        grid=(N // blk,),
        in_specs=[
            pl.BlockSpec((blk, 10), lambda i: (i, 0)),
            pl.BlockSpec((blk, 175), lambda i: (i, 0)),
            pl.BlockSpec((10, 256), lambda i: (0, 0)),
            pl.BlockSpec((256,), lambda i: (0,)),
            pl.BlockSpec((256, 10), lambda i: (0, 0)),
            pl.BlockSpec((10,), lambda i: (0,)),
            pl.BlockSpec((10,), lambda i: (0,)),
            pl.BlockSpec((10,), lambda i: (0,)),
            pl.BlockSpec((10, 64), lambda i: (0, 0)),
            pl.BlockSpec((64,), lambda i: (0,)),
            pl.BlockSpec((64,), lambda i: (0,)),
            pl.BlockSpec((64,), lambda i: (0,)),
            pl.BlockSpec((64, 64), lambda i: (0, 0)),
            pl.BlockSpec((64,), lambda i: (0,)),
            pl.BlockSpec((64,), lambda i: (0,)),
            pl.BlockSpec((64,), lambda i: (0,)),
            pl.BlockSpec((175, 400), lambda i: (0, 0)),
            pl.BlockSpec((400,), lambda i: (0,)),
            pl.BlockSpec((400, 800), lambda i: (0, 0)),
            pl.BlockSpec((800,), lambda i: (0,)),
            pl.BlockSpec((800, 1600), lambda i: (0, 0)),
            pl.BlockSpec((1600,), lambda i: (0,)),
            pl.BlockSpec((1600, 64), lambda i: (0, 0)),
        ],
        out_specs=pl.BlockSpec((blk, 128), lambda i: (i, 0)),
        out_shape=jax.ShapeDtypeStruct((N, 128), jnp.float32),
    )(node_x, xg, p['ma_w1'], p['ma_b1'], p['ma_w2'], p['ma_b2'], p['ma_g'],
      p['ma_be'], p['na_w1'], p['na_b1'], p['na_g1'], p['na_be1'], p['na_w2'],
      p['na_b2'], p['na_g2'], p['na_be2'], bw1, bb1, bw2, bb2, bw3, bb3,
      jnp.asarray(_POOL_P))


def _edge_enc_kernel(x_ref, w1_ref, b1_ref, g1_ref, be1_ref, w2_ref, b2_ref,
                     g2_ref, be2_ref, out_ref):
    x = x_ref[...]
    h = jnp.maximum(_ln(jnp.dot(x, w1_ref[...],
                                preferred_element_type=jnp.float32)
                        + b1_ref[...], g1_ref[...], be1_ref[...]), 0.0)
    h = _mish(_ln(jnp.dot(h, w2_ref[...], preferred_element_type=jnp.float32)
                  + b2_ref[...], g2_ref[...], be2_ref[...]))
    out_ref[...] = jnp.concatenate([h, jnp.zeros_like(h)], axis=1)


def _edge_enc(edge_x, p):
    blk = 1000
    return pl.pallas_call(
        _edge_enc_kernel,
        grid=(E // blk,),
        in_specs=[
            pl.BlockSpec((blk, 12), lambda i: (i, 0)),
            pl.BlockSpec((12, 64), lambda i: (0, 0)),
            pl.BlockSpec((64,), lambda i: (0,)),
            pl.BlockSpec((64,), lambda i: (0,)),
            pl.BlockSpec((64,), lambda i: (0,)),
            pl.BlockSpec((64, 64), lambda i: (0, 0)),
            pl.BlockSpec((64,), lambda i: (0,)),
            pl.BlockSpec((64,), lambda i: (0,)),
            pl.BlockSpec((64,), lambda i: (0,)),
        ],
        out_specs=pl.BlockSpec((blk, 128), lambda i: (i, 0)),
        out_shape=jax.ShapeDtypeStruct((E, 128), jnp.float32),
    )(edge_x, p['ea_w1'], p['ea_b1'], p['ea_g1'], p['ea_be1'], p['ea_w2'],
      p['ea_b2'], p['ea_g2'], p['ea_be2'])


def _mid_kernel(hs_ref, hd_ref, e_ref, ws_ref, wd_ref, we_ref, bs_ref,
                m_ref, enew_ref):
    e = e_ref[...][:, :64]
    t = (jnp.dot(hs_ref[...], ws_ref[...], preferred_element_type=jnp.float32)
         + jnp.dot(hd_ref[...], wd_ref[...], preferred_element_type=jnp.float32)
         + jnp.dot(e, we_ref[...], preferred_element_type=jnp.float32)
         + bs_ref[...])
    m_ref[...] = _mish(t[:, :128])
    en = e + _mish(t[:, 128:])
    enew_ref[...] = jnp.concatenate([en, jnp.zeros_like(en)], axis=1)


def _mid(g1, g2, ep, ws, wd, we, bs):
    blk = 512
    return pl.pallas_call(
        _mid_kernel,
        grid=(E_ALLOC // blk,),
        in_specs=[
            pl.BlockSpec((blk, 128), lambda i: (i, 0)),
            pl.BlockSpec((blk, 128), lambda i: (i, 0)),
            pl.BlockSpec((blk, 128), lambda i: (i, 0)),
            pl.BlockSpec((128, 192), lambda i: (0, 0)),
            pl.BlockSpec((128, 192), lambda i: (0, 0)),
            pl.BlockSpec((64, 192), lambda i: (0, 0)),
            pl.BlockSpec((192,), lambda i: (0,)),
        ],
        out_specs=[
            pl.BlockSpec((blk, 128), lambda i: (i, 0)),
            pl.BlockSpec((blk, 128), lambda i: (i, 0)),
        ],
        out_shape=[
            jax.ShapeDtypeStruct((E_ALLOC, 128), jnp.float32),
            jax.ShapeDtypeStruct((E_ALLOC, 128), jnp.float32),
        ],
    )(g1, g2, ep, ws, wd, we, bs)


def _update_kernel(h_ref, agg_ref, wh_ref, wa_ref, b_ref, g_ref, be_ref,
                   out_ref):
    h = h_ref[...]
    u = (jnp.dot(h, wh_ref[...], preferred_element_type=jnp.float32)
         + jnp.dot(agg_ref[...], wa_ref[...], preferred_element_type=jnp.float32)
         + b_ref[...])
    out_ref[...] = h + _mish(_ln(u, g_ref[...], be_ref[...]))


def _update(hcur, agg, wh, wa, bb, g, be):
    blk = 1000
    return pl.pallas_call(
        _update_kernel,
        grid=(N // blk,),
        in_specs=[
            pl.BlockSpec((blk, 128), lambda i: (i, 0)),
            pl.BlockSpec((blk, 128), lambda i: (i, 0)),
            pl.BlockSpec((128, 128), lambda i: (0, 0)),
            pl.BlockSpec((128, 128), lambda i: (0, 0)),
            pl.BlockSpec((128,), lambda i: (0,)),
            pl.BlockSpec((128,), lambda i: (0,)),
            pl.BlockSpec((128,), lambda i: (0,)),
        ],
        out_specs=pl.BlockSpec((blk, 128), lambda i: (i, 0)),
        out_shape=jax.ShapeDtypeStruct((N, 128), jnp.float32),
    )(hcur, agg, wh, wa, bb, g, be)


def _pool_kernel(h_ref, out_ref):
    out_ref[...] = (jnp.sum(h_ref[...], axis=0, keepdims=True)
                    * (1.0 / (N // B)))[None]


def _pool(node_emb):
    return pl.pallas_call(
        _pool_kernel,
        grid=(B,),
        in_specs=[pl.BlockSpec((N // B, 128), lambda i: (i, 0))],
        out_specs=pl.BlockSpec((1, 1, 128), lambda i: (i, 0, 0)),
        out_shape=jax.ShapeDtypeStruct((B, 1, 128), jnp.float32),
    )(node_emb)


def _head_kernel(ne_ref, ge_ref, w1_ref, b1_ref, g_ref, be_ref, w2_ref, b2_ref,
                 out_ref):
    ne = ne_ref[...]
    ge = jnp.broadcast_to(ge_ref[0], (ne.shape[0], 128))
    u = (jnp.dot(ne, w1_ref[:128], preferred_element_type=jnp.float32)
         + jnp.dot(ge, w1_ref[128:], preferred_element_type=jnp.float32)
         + b1_ref[...])
    sx = _mish(_ln(u, g_ref[...], be_ref[...]))
    out_ref[...] = (jnp.dot(sx, w2_ref[...], preferred_element_type=jnp.float32)
                    + b2_ref[...])


def _head(node_emb, graph_emb, w1, b1, g, be, w2, b2):
    blk = N // B
    w2p = jnp.zeros((256, 128), jnp.float32).at[:, :NUM_CLASSES].set(w2)
    b2p = jnp.zeros((128,), jnp.float32).at[:NUM_CLASSES].set(b2)
    out = pl.pallas_call(
        _head_kernel,
        grid=(B,),
        in_specs=[
            pl.BlockSpec((blk, 128), lambda i: (i, 0)),
            pl.BlockSpec((1, 1, 128), lambda i: (i, 0, 0)),
            pl.BlockSpec((256, 256), lambda i: (0, 0)),
            pl.BlockSpec((256,), lambda i: (0,)),
            pl.BlockSpec((256,), lambda i: (0,)),
            pl.BlockSpec((256,), lambda i: (0,)),
            pl.BlockSpec((256, 128), lambda i: (0, 0)),
            pl.BlockSpec((128,), lambda i: (0,)),
        ],
        out_specs=pl.BlockSpec((blk, 128), lambda i: (i, 0)),
        out_shape=jax.ShapeDtypeStruct((N, 128), jnp.float32),
    )(node_emb, graph_emb, w1, b1, g, be, w2p, b2p)
    return out[:, :NUM_CLASSES]


def _pad_cols(w, total):
    return jnp.concatenate([w, jnp.zeros((w.shape[0], total - w.shape[1]),
                                         w.dtype)], axis=1)


def kernel(node_x, node_grid, edge_x, edge_index, batch_num_nodes, params):
    p = params
    bw1, bb1 = _conv_as_matmul_weights(p['c1_w'], p['c1_b'], p['bn1_g'],
                                       p['bn1_b'])
    bw2, bb2 = _conv_as_matmul_weights(p['c2_w'], p['c2_b'], p['bn2_g'],
                                       p['bn2_b'])
    bw3, bb3 = _conv_as_matmul_weights(p['c3_w'], p['c3_b'], p['bn3_g'],
                                       p['bn3_b'])
    node_feat = _node_enc(node_x, node_grid.reshape(N, 175), p,
                          bw1, bb1, bw2, bb2, bw3, bb3)
    e0 = _edge_enc(edge_x, p)
    src = edge_index[0]
    dst = edge_index[1]
    idr, srr, dsr, lens_flat = _sc_bucket(src, dst)
    lens = lens_flat.reshape(NB, 16)[:, 0]
    lens_pad = ((lens + PCH - 1) // PCH) * PCH
    offs = jnp.concatenate([jnp.zeros((1,), jnp.int32),
                            jnp.cumsum(lens_pad)]).astype(jnp.int32)[:NB]
    srcp, dstp, ep = _sc_permute(idr, srr, dsr, lens, offs, e0)
    hcur = node_feat
    for l in range(NUM_LAYERS):
        mw, mb = p['msg_w%d' % l], p['msg_b%d' % l]
        ew, eb = p['edg_w%d' % l], p['edg_b%d' % l]
        ws = jnp.concatenate([mw[:128], ew[:128]], axis=1)
        wd = jnp.concatenate([mw[128:256], ew[128:256]], axis=1)
        we = jnp.concatenate([mw[256:], ew[256:]], axis=1)
        bs = jnp.concatenate([mb, eb])
        g1, g2 = _sc_gather(hcur, srcp, dstp, lens, offs)
        m, ep = _mid(g1, g2, ep, ws, wd, we, bs)
        agg = _sc_scatter(m, dstp, lens, offs)
        uw, ub = p['upd_w%d' % l], p['upd_b%d' % l]
        hcur = _update(hcur, agg[:N], uw[:128], uw[128:], ub,
                       p['uln_g%d' % l], p['uln_b%d' % l])
    node_emb = hcur
    graph_emb = _pool(node_emb)
    seg = _head(node_emb, graph_emb, p['sh_w1'], p['sh_b1'], p['sh_g'],
                p['sh_be'], p['sh_w2'], p['sh_b2'])
    return seg, node_emb


# speedup vs baseline: 1.1986x; 1.0289x over previous
import functools

import jax
import jax.numpy as jnp
import numpy as np
from jax import lax
from jax.experimental import pallas as pl
from jax.experimental.pallas import tpu as pltpu
from jax.experimental.pallas import tpu_sc as plsc

N = 50000
E = 800000
B = 50
NUM_LAYERS = 2
NUM_CLASSES = 25

NC = 2
NS = 16
NW = NC * NS
NB = 128
BW = 391
NBW = NB * BW
BPW = NB // NW
CAP = E + 4096
PCH = 512
GCH = 128
SCH = 256
E_ALLOC = 866304

_MESH = dict(core_axis_name="c", subcore_axis_name="s")
_SC_PARAMS = pltpu.CompilerParams(needs_layout_passes=False)


def _mish(x):
    return x * jnp.tanh(jax.nn.softplus(x))


def _ln(x, g, b, eps=1e-5):
    mu = jnp.mean(x, axis=-1, keepdims=True)
    var = jnp.mean((x - mu) ** 2, axis=-1, keepdims=True)
    return (x - mu) / jnp.sqrt(var + eps) * g + b


def _conv_mask():
    M = np.zeros((9, 25, 25), np.float32)
    for di in range(3):
        for dj in range(3):
            k = di * 3 + dj
            for pi in range(5):
                for pj in range(5):
                    qi, qj = pi + di - 1, pj + dj - 1
                    if 0 <= qi < 5 and 0 <= qj < 5:
                        M[k, qi * 5 + qj, pi * 5 + pj] = 1.0
    return M


_CONV_M = _conv_mask()
_POOL_P = np.kron(np.eye(64, dtype=np.float32),
                  np.full((25, 1), 1.0 / 25, np.float32))


def _conv_as_matmul_weights(w, b, g, beta):
    co, ci = w.shape[0], w.shape[1]
    wf = (w * g[:, None, None, None]).reshape(co, ci, 9)
    bias = b * g + beta
    big = jnp.einsum('oik,kqp->iqop', wf, jnp.asarray(_CONV_M))
    return big.reshape(ci * 25, co * 25), jnp.repeat(bias, 25)


def _iota16():
    return lax.iota(jnp.int32, 16)


def _al(x, n=8):
    return pl.multiple_of(x, n)


def _vextract(vbuf, ref_idx):
    idxv = jnp.full((16,), ref_idx, jnp.int32)
    return plsc.load_gather(vbuf, [idxv])[0]


def _sc_bucket_body(src_hbm, dst_hbm, idr_hbm, srr_hbm, dsr_hbm, lens_hbm,
                    dbuf0, dbuf1, sbuf0, sbuf1, oid, osr, ods, idsc, srcsc,
                    dstsc, lbuf, scur, semd0, semd1, sems0, sems1):
    c = lax.axis_index("c")
    s = lax.axis_index("s")
    w = s * NC + c
    CHB = 2000
    NCH = E // CHB
    lo0 = w * (BPW * BW)
    hi0 = lo0 + BPW * BW
    dbufs = (dbuf0, dbuf1)
    sbufs = (sbuf0, sbuf1)
    semds = (semd0, semd1)
    semss = (sems0, sems1)
    for j in range(BPW):
        scur[j] = 0
        scur[BPW + j] = 0

    def start(ci, u):
        pltpu.async_copy(dst_hbm.at[pl.ds(ci * CHB, CHB)], dbufs[u], semds[u])
        pltpu.async_copy(src_hbm.at[pl.ds(ci * CHB, CHB)], sbufs[u], semss[u])

    def wait(ci, u):
        pltpu.make_async_copy(dst_hbm.at[pl.ds(ci * CHB, CHB)], dbufs[u],
                              semds[u]).wait()
        pltpu.make_async_copy(src_hbm.at[pl.ds(ci * CHB, CHB)], sbufs[u],
                              semss[u]).wait()

    def process(ci, u):
        dbuf = dbufs[u]
        sbuf = sbufs[u]

        def vreg_body(i, _):
            v = dbuf[pl.ds(i * 16, 16)]
            m = (v >= lo0) & (v < hi0)
            cnt = plsc.all_reduce_population_count(m)[0]

            @pl.when(cnt > 0)
            def _extract():
                idsc[...] = (ci * CHB + i * 16) + _iota16()
                dstsc[...] = v
                srcsc[...] = sbuf[pl.ds(i * 16, 16)]

                def match_body(k, mvec):
                    fv = plsc.all_reduce_ffs(mvec)
                    idsp = plsc.load_gather(idsc, [fv])
                    ssp = plsc.load_gather(srcsc, [fv])
                    dsp = plsc.load_gather(dstsc, [fv])
                    d2 = dsp[0] - lo0
                    jj = ((d2 >= BW).astype(jnp.int32)
                          + (d2 >= 2 * BW).astype(jnp.int32)
                          + (d2 >= 3 * BW).astype(jnp.int32))
                    posj = scur[jj]
                    tgt = jnp.full((16,), jj * 4096 + posj, jnp.int32)
                    plsc.store_scatter(oid, [tgt], idsp)
                    plsc.store_scatter(osr, [tgt], ssp)
                    plsc.store_scatter(ods, [tgt], dsp)
                    scur[jj] = posj + 1
                    return mvec & (_iota16() != fv)

                lax.fori_loop(0, cnt, match_body, m)

            return 0

        lax.fori_loop(0, CHB // 16, vreg_body, 0)
        for j in range(BPW):
            @pl.when(scur[j] >= 2048)
            def _flush(j=j):
                base = scur[BPW + j]
                hb = _al((w * BPW + j) * CAP + base)
                pltpu.sync_copy(oid.at[pl.ds(j * 4096, 2048)],
                                idr_hbm.at[pl.ds(hb, 2048)])
                pltpu.sync_copy(osr.at[pl.ds(j * 4096, 2048)],
                                srr_hbm.at[pl.ds(hb, 2048)])
                pltpu.sync_copy(ods.at[pl.ds(j * 4096, 2048)],
                                dsr_hbm.at[pl.ds(hb, 2048)])
                nt = scur[j] - 2048

                def mv(k, _):
                    d0 = _al(j * 4096 + k * 16, 16)
                    d1 = _al(j * 4096 + 2048 + k * 16, 16)
                    oid[pl.ds(d0, 16)] = oid[pl.ds(d1, 16)]
                    osr[pl.ds(d0, 16)] = osr[pl.ds(d1, 16)]
                    ods[pl.ds(d0, 16)] = ods[pl.ds(d1, 16)]
                    return 0

                lax.fori_loop(0, (nt + 15) >> 4, mv, 0)
                scur[j] = nt
                scur[BPW + j] = base + 2048

    start(0, 0)

    def pair_body(cj, _):
        ci0 = cj * 2
        start(ci0 + 1, 1)
        wait(ci0, 0)
        process(ci0, 0)

        @pl.when(ci0 + 2 < NCH)
        def _s2():
            start(ci0 + 2, 0)

        wait(ci0 + 1, 1)
        process(ci0 + 1, 1)
        return 0

    lax.fori_loop(0, NCH // 2, pair_body, 0)
    for j in range(BPW):
        hb = _al((w * BPW + j) * CAP + scur[BPW + j])
        pltpu.sync_copy(oid.at[pl.ds(j * 4096, 2048)], idr_hbm.at[pl.ds(hb, 2048)])
        pltpu.sync_copy(osr.at[pl.ds(j * 4096, 2048)], srr_hbm.at[pl.ds(hb, 2048)])
        pltpu.sync_copy(ods.at[pl.ds(j * 4096, 2048)], dsr_hbm.at[pl.ds(hb, 2048)])
        lbuf[pl.ds(j * 16, 16)] = jnp.full((16,), scur[BPW + j] + scur[j],
                                           jnp.int32)
    pltpu.sync_copy(lbuf, lens_hbm.at[pl.ds(_al(w * BPW * 16), BPW * 16)])


def _sc_bucket(src, dst):
    return pl.kernel(
        _sc_bucket_body,
        out_type=[
            jax.ShapeDtypeStruct((NB * CAP,), jnp.int32),
            jax.ShapeDtypeStruct((NB * CAP,), jnp.int32),
            jax.ShapeDtypeStruct((NB * CAP,), jnp.int32),
            jax.ShapeDtypeStruct((NB * 16,), jnp.int32),
        ],
        mesh=plsc.VectorSubcoreMesh(**_MESH),
        compiler_params=_SC_PARAMS,
        scratch_types=[
            pltpu.VMEM((2000,), jnp.int32),
            pltpu.VMEM((2000,), jnp.int32),
            pltpu.VMEM((2000,), jnp.int32),
            pltpu.VMEM((2000,), jnp.int32),
            pltpu.VMEM((BPW * 4096,), jnp.int32),
            pltpu.VMEM((BPW * 4096,), jnp.int32),
            pltpu.VMEM((BPW * 4096,), jnp.int32),
            pltpu.VMEM((16,), jnp.int32),
            pltpu.VMEM((16,), jnp.int32),
            pltpu.VMEM((16,), jnp.int32),
            pltpu.VMEM((BPW * 16,), jnp.int32),
            pltpu.SMEM((2 * BPW,), jnp.int32),
            pltpu.SemaphoreType.DMA,
            pltpu.SemaphoreType.DMA,
            pltpu.SemaphoreType.DMA,
            pltpu.SemaphoreType.DMA,
        ],
    )(src, dst)


def _sc_permute_body(idr_hbm, srr_hbm, dsr_hbm, lens_hbm, offs_hbm, e0_hbm,
                     srcp_hbm, dstp_hbm, e0p_hbm,
                     ibuf, svbuf, dvbuf, ebuf, lvbuf, ovbuf, sem):
    c = lax.axis_index("c")
    s = lax.axis_index("s")
    w = s * NC + c
    pltpu.sync_copy(lens_hbm, lvbuf)
    pltpu.sync_copy(offs_hbm, ovbuf)
    for j in range(BPW):
        b = w * BPW + j
        lo = b * BW
        n = _vextract(lvbuf, b)
        off = _vextract(ovbuf, b)
        nch = (n + PCH - 1) >> 9

        def chunk_body(ci, _):
            rb = _al(b * CAP + ci * PCH)
            pltpu.sync_copy(idr_hbm.at[pl.ds(rb, PCH)], ibuf)
            pltpu.sync_copy(srr_hbm.at[pl.ds(rb, PCH)], svbuf)
            pltpu.sync_copy(dsr_hbm.at[pl.ds(rb, PCH)], dvbuf)

            def sanitize(i, _):
                g = ci * PCH + i * 16 + _iota16()
                ok = g < n
                ibuf[pl.ds(i * 16, 16)] = jnp.where(ok, ibuf[pl.ds(i * 16, 16)], 0)
                svbuf[pl.ds(i * 16, 16)] = jnp.where(ok, svbuf[pl.ds(i * 16, 16)],
                                                     0)
                dvbuf[pl.ds(i * 16, 16)] = jnp.where(ok, dvbuf[pl.ds(i * 16, 16)],
                                                     lo + BW)
                return 0

            lax.fori_loop(0, PCH // 16, sanitize, 0)
            pltpu.async_copy(e0_hbm.at[ibuf], ebuf, sem).wait()
            ob = _al(off + ci * PCH)
            pltpu.sync_copy(svbuf, srcp_hbm.at[pl.ds(ob, PCH)])
            pltpu.sync_copy(dvbuf, dstp_hbm.at[pl.ds(ob, PCH)])
            pltpu.sync_copy(ebuf, e0p_hbm.at[pl.ds(ob, PCH)])
            return 0

        lax.fori_loop(0, nch, chunk_body, 0)


def _sc_permute(idr, srr, dsr, lens, offs, e0):
    return pl.kernel(
        _sc_permute_body,
        out_type=[
            jax.ShapeDtypeStruct((E_ALLOC,), jnp.int32),
            jax.ShapeDtypeStruct((E_ALLOC,), jnp.int32),
            jax.ShapeDtypeStruct((E_ALLOC, 128), jnp.float32),
        ],
        mesh=plsc.VectorSubcoreMesh(**_MESH),
        compiler_params=_SC_PARAMS,
        scratch_types=[
            pltpu.VMEM((PCH,), jnp.int32),
            pltpu.VMEM((PCH,), jnp.int32),
            pltpu.VMEM((PCH,), jnp.int32),
            pltpu.VMEM((PCH, 128), jnp.float32),
            pltpu.VMEM((NB,), jnp.int32),
            pltpu.VMEM((NB,), jnp.int32),
            pltpu.SemaphoreType.DMA,
        ],
    )(idr, srr, dsr, lens, offs, e0)


def _sc_gather_body(h_hbm, srcp_hbm, dstp_hbm, lens_hbm, offs_hbm,
                    g1_hbm, g2_hbm, sib0, sib1, dib0, dib1, g1b0, g1b1,
                    g2b0, g2b1, lvbuf, ovbuf,
                    smg10, smg11, smg20, smg21, smw10, smw11, smw20, smw21):
    c = lax.axis_index("c")
    s = lax.axis_index("s")
    w = s * NC + c
    pltpu.sync_copy(lens_hbm, lvbuf)
    pltpu.sync_copy(offs_hbm, ovbuf)
    sibs = (sib0, sib1)
    dibs = (dib0, dib1)
    g1bs = (g1b0, g1b1)
    g2bs = (g2b0, g2b1)
    smg1 = (smg10, smg11)
    smg2 = (smg20, smg21)
    smw1 = (smw10, smw11)
    smw2 = (smw20, smw21)
    for j in range(BPW):
        b = w * BPW + j
        n = _vextract(lvbuf, b)
        off = _vextract(ovbuf, b)
        npad = ((n + PCH - 1) >> 9) << 9
        nch = npad // GCH

        def issue(ci, u):
            base = _al(off + ci * GCH)
            pltpu.sync_copy(srcp_hbm.at[pl.ds(base, GCH)], sibs[u])
            pltpu.sync_copy(dstp_hbm.at[pl.ds(base, GCH)], dibs[u])

            def clampv(i, _):
                dibs[u][pl.ds(i * 16, 16)] = jnp.minimum(
                    dibs[u][pl.ds(i * 16, 16)], N - 1)
                return 0

            lax.fori_loop(0, GCH // 16, clampv, 0)
            pltpu.async_copy(h_hbm.at[sibs[u]], g1bs[u], smg1[u])
            pltpu.async_copy(h_hbm.at[dibs[u]], g2bs[u], smg2[u])

        def finish(ci, u):
            base = _al(off + ci * GCH)
            pltpu.make_async_copy(h_hbm.at[sibs[u]], g1bs[u], smg1[u]).wait()
            pltpu.make_async_copy(h_hbm.at[dibs[u]], g2bs[u], smg2[u]).wait()
            pltpu.async_copy(g1bs[u], g1_hbm.at[pl.ds(base, GCH)], smw1[u])
            pltpu.async_copy(g2bs[u], g2_hbm.at[pl.ds(base, GCH)], smw2[u])

        def wait_wb(ci, u):
            base = _al(off + ci * GCH)
            pltpu.make_async_copy(g1bs[u], g1_hbm.at[pl.ds(base, GCH)],
                                  smw1[u]).wait()
            pltpu.make_async_copy(g2bs[u], g2_hbm.at[pl.ds(base, GCH)],
                                  smw2[u]).wait()

        @pl.when(nch > 0)
        def _run():
            issue(0, 0)

            def pair_body(cj, _):
                ci0 = cj * 2

                @pl.when(cj > 0)
                def _w1():
                    wait_wb(ci0 - 1, 1)

                issue(ci0 + 1, 1)
                finish(ci0, 0)

                @pl.when(ci0 + 2 < nch)
                def _n0():
                    wait_wb(ci0, 0)
                    issue(ci0 + 2, 0)

                finish(ci0 + 1, 1)
                return 0

            lax.fori_loop(0, nch // 2, pair_body, 0)
            wait_wb(nch - 2, 0)
            wait_wb(nch - 1, 1)


def _sc_gather(hcur, srcp, dstp, lens, offs):
    return pl.kernel(
        _sc_gather_body,
        out_type=[
            jax.ShapeDtypeStruct((E_ALLOC, 128), jnp.float32),
            jax.ShapeDtypeStruct((E_ALLOC, 128), jnp.float32),
        ],
        mesh=plsc.VectorSubcoreMesh(**_MESH),
        compiler_params=_SC_PARAMS,
        scratch_types=[
            pltpu.VMEM((GCH,), jnp.int32),
            pltpu.VMEM((GCH,), jnp.int32),
            pltpu.VMEM((GCH,), jnp.int32),
            pltpu.VMEM((GCH,), jnp.int32),
            pltpu.VMEM((GCH, 128), jnp.float32),
            pltpu.VMEM((GCH, 128), jnp.float32),
            pltpu.VMEM((GCH, 128), jnp.float32),
            pltpu.VMEM((GCH, 128), jnp.float32),
            pltpu.VMEM((NB,), jnp.int32),
            pltpu.VMEM((NB,), jnp.int32),
        ] + [pltpu.SemaphoreType.DMA] * 8,
    )(hcur, srcp, dstp, lens, offs)


def _sc_scatter_body(m_hbm, dstp_hbm, lens_hbm, offs_hbm, agg_hbm,
                     mbuf0, mbuf1, dbuf0, dbuf1, aggbuf, lvbuf, ovbuf,
                     sem0, sem1, semd0, semd1):
    c = lax.axis_index("c")
    s = lax.axis_index("s")
    w = s * NC + c
    pltpu.sync_copy(lens_hbm, lvbuf)
    pltpu.sync_copy(offs_hbm, ovbuf)
    zeros = jnp.zeros((16,), jnp.float32)
    mbufs = (mbuf0, mbuf1)
    dbufs = (dbuf0, dbuf1)
    sems = (sem0, sem1)
    semds = (semd0, semd1)

    for j in range(BPW):
        b = w * BPW + j
        lo = b * BW
        n = _vextract(lvbuf, b)
        off = _vextract(ovbuf, b)
        npad = ((n + PCH - 1) >> 9) << 9
        nch = npad // SCH

        def zero_body(k, _):
            base = _al(k * 256, 16)
            for u in range(16):
                aggbuf[pl.ds(base + u * 16, 16)] = zeros
            return 0

        lax.fori_loop(0, (BW + 1) * 128 // 256, zero_body, 0)

        def start(ci, u):
            base = _al(off + ci * SCH)
            return (pltpu.async_copy(m_hbm.at[pl.ds(_al(base * 128), SCH * 128)],
                                     mbufs[u], sems[u]),
                    pltpu.async_copy(dstp_hbm.at[pl.ds(base, SCH)],
                                     dbufs[u], semds[u]))

        def wait(ci, u):
            base = _al(off + ci * SCH)
            pltpu.make_async_copy(m_hbm.at[pl.ds(_al(base * 128), SCH * 128)],
                                  mbufs[u], sems[u]).wait()
            pltpu.make_async_copy(dstp_hbm.at[pl.ds(base, SCH)],
                                  dbufs[u], semds[u]).wait()

        def process(u):
            mbuf = mbufs[u]
            dbuf = dbufs[u]

            def grp_body(g, _):
                rows = (g * 16 + _iota16()) * 128
                dv = dbuf[pl.ds(g * 16, 16)]
                doff = jnp.clip(dv - lo, 0, BW) * 128
                for cc in range(16):
                    vs = [plsc.load_gather(mbuf, [rows + cc * 8 + u])
                          for u in range(8)]
                    for u in range(8):
                        plsc.addupdate_scatter(aggbuf, [doff + cc * 8 + u],
                                               vs[u])
                return 0

            lax.fori_loop(0, SCH // 16, grp_body, 0)

        @pl.when(nch > 0)
        def _run():
            start(0, 0)

            def pair_body(cj, _):
                ci0 = cj * 2

                @pl.when(ci0 + 1 < nch)
                def _s1():
                    start(ci0 + 1, 1)

                wait(ci0, 0)
                process(0)

                @pl.when(ci0 + 2 < nch)
                def _s2():
                    start(ci0 + 2, 0)

                @pl.when(ci0 + 1 < nch)
                def _p1():
                    wait(ci0 + 1, 1)
                    process(1)

                return 0

            lax.fori_loop(0, (nch + 1) // 2, pair_body, 0)

        pltpu.sync_copy(aggbuf.at[pl.ds(0, BW * 128)],
                        agg_hbm.at[pl.ds(_al(lo * 128), BW * 128)])


def _sc_scatter(m, dstp, lens, offs):
    m_flat = m.reshape(E_ALLOC * 128)
    out = pl.kernel(
        _sc_scatter_body,
        out_type=jax.ShapeDtypeStruct((NBW * 128,), jnp.float32),
        mesh=plsc.VectorSubcoreMesh(**_MESH),
        compiler_params=_SC_PARAMS,
        scratch_types=[
            pltpu.VMEM((SCH * 128,), jnp.float32),
            pltpu.VMEM((SCH * 128,), jnp.float32),
            pltpu.VMEM((SCH,), jnp.int32),
            pltpu.VMEM((SCH,), jnp.int32),
            pltpu.VMEM(((BW + 1) * 128,), jnp.float32),
            pltpu.VMEM((NB,), jnp.int32),
            pltpu.VMEM((NB,), jnp.int32),
            pltpu.SemaphoreType.DMA,
            pltpu.SemaphoreType.DMA,
            pltpu.SemaphoreType.DMA,
            pltpu.SemaphoreType.DMA,
        ],
    )(m_flat, dstp, lens, offs)
    return out.reshape(NBW, 128)


def _node_enc_kernel(nx_ref, xg_ref, w1_ref, b1_ref, w2_ref, b2_ref, mg_ref,
                     mbe_ref, nw1_ref, nb1_ref, ng1_ref, nbe1_ref, nw2_ref,
                     nb2_ref, ng2_ref, nbe2_ref, cw1_ref, cb1_ref, cw2_ref,
                     cb2_ref, cw3_ref, cb3_ref, pp_ref, out_ref):
    x = nx_ref[...]
    hid = jnp.maximum(jnp.dot(x, w1_ref[...], preferred_element_type=jnp.float32)
                      + b1_ref[...], 0.0)
    ma = (jnp.dot(hid, w2_ref[...], preferred_element_type=jnp.float32)
          + b2_ref[...]) * mg_ref[...] + mbe_ref[...]
    h = jnp.maximum(_ln(jnp.dot(ma, nw1_ref[...],
                                preferred_element_type=jnp.float32)
                        + nb1_ref[...], ng1_ref[...], nbe1_ref[...]), 0.0)
    h = _mish(_ln(jnp.dot(h, nw2_ref[...], preferred_element_type=jnp.float32)
                  + nb2_ref[...], ng2_ref[...], nbe2_ref[...]))
    y = _mish(jnp.dot(xg_ref[...], cw1_ref[...],
                      preferred_element_type=jnp.float32) + cb1_ref[...])
    y = _mish(jnp.dot(y, cw2_ref[...], preferred_element_type=jnp.float32)
              + cb2_ref[...])
    y = _mish(jnp.dot(y, cw3_ref[...], preferred_element_type=jnp.float32)
              + cb3_ref[...])
    g = jnp.dot(y, pp_ref[...], preferred_element_type=jnp.float32)
    out_ref[...] = jnp.concatenate([h, g], axis=1)


def _node_enc(node_x, xg, p, bw1, bb1, bw2, bb2, bw3, bb3):
    blk = 400
    return pl.pallas_call(
        _node_enc_kernel,
        grid=(N // blk,),
        in_specs=[
            pl.BlockSpec((blk, 10), lambda i: (i, 0)),
            pl.BlockSpec((blk, 175), lambda i: (i, 0)),
            pl.BlockSpec((10, 256), lambda i: (0, 0)),
            pl.BlockSpec((256,), lambda i: (0,)),
            pl.BlockSpec((256, 10), lambda i: (0, 0)),
            pl.BlockSpec((10,), lambda i: (0,)),
            pl.BlockSpec((10,), lambda i: (0,)),
            pl.BlockSpec((10,), lambda i: (0,)),
            pl.BlockSpec((10, 64), lambda i: (0, 0)),
            pl.BlockSpec((64,), lambda i: (0,)),
            pl.BlockSpec((64,), lambda i: (0,)),
            pl.BlockSpec((64,), lambda i: (0,)),
            pl.BlockSpec((64, 64), lambda i: (0, 0)),
            pl.BlockSpec((64,), lambda i: (0,)),
            pl.BlockSpec((64,), lambda i: (0,)),
            pl.BlockSpec((64,), lambda i: (0,)),
            pl.BlockSpec((175, 400), lambda i: (0, 0)),
            pl.BlockSpec((400,), lambda i: (0,)),
            pl.BlockSpec((400, 800), lambda i: (0, 0)),
            pl.BlockSpec((800,), lambda i: (0,)),
            pl.BlockSpec((800, 1600), lambda i: (0, 0)),
            pl.BlockSpec((1600,), lambda i: (0,)),
            pl.BlockSpec((1600, 64), lambda i: (0, 0)),
        ],
        out_specs=pl.BlockSpec((blk, 128), lambda i: (i, 0)),
        out_shape=jax.ShapeDtypeStruct((N, 128), jnp.float32),
    )(node_x, xg, p['ma_w1'], p['ma_b1'], p['ma_w2'], p['ma_b2'], p['ma_g'],
      p['ma_be'], p['na_w1'], p['na_b1'], p['na_g1'], p['na_be1'], p['na_w2'],
      p['na_b2'], p['na_g2'], p['na_be2'], bw1, bb1, bw2, bb2, bw3, bb3,
      jnp.asarray(_POOL_P))


def _edge_enc_kernel(x_ref, w1_ref, b1_ref, g1_ref, be1_ref, w2_ref, b2_ref,
                     g2_ref, be2_ref, out_ref):
    x = x_ref[...]
    h = jnp.maximum(_ln(jnp.dot(x, w1_ref[...],
                                preferred_element_type=jnp.float32)
                        + b1_ref[...], g1_ref[...], be1_ref[...]), 0.0)
    h = _mish(_ln(jnp.dot(h, w2_ref[...], preferred_element_type=jnp.float32)
                  + b2_ref[...], g2_ref[...], be2_ref[...]))
    out_ref[...] = jnp.concatenate([h, jnp.zeros_like(h)], axis=1)


def _edge_enc(edge_x, p):
    blk = 1000
    return pl.pallas_call(
        _edge_enc_kernel,
        grid=(E // blk,),
        in_specs=[
            pl.BlockSpec((blk, 12), lambda i: (i, 0)),
            pl.BlockSpec((12, 64), lambda i: (0, 0)),
            pl.BlockSpec((64,), lambda i: (0,)),
            pl.BlockSpec((64,), lambda i: (0,)),
            pl.BlockSpec((64,), lambda i: (0,)),
            pl.BlockSpec((64, 64), lambda i: (0, 0)),
            pl.BlockSpec((64,), lambda i: (0,)),
            pl.BlockSpec((64,), lambda i: (0,)),
            pl.BlockSpec((64,), lambda i: (0,)),
        ],
        out_specs=pl.BlockSpec((blk, 128), lambda i: (i, 0)),
        out_shape=jax.ShapeDtypeStruct((E, 128), jnp.float32),
    )(edge_x, p['ea_w1'], p['ea_b1'], p['ea_g1'], p['ea_be1'], p['ea_w2'],
      p['ea_b2'], p['ea_g2'], p['ea_be2'])


def _mid_kernel(hs_ref, hd_ref, e_ref, ws_ref, wd_ref, we_ref, bs_ref,
                m_ref, enew_ref):
    e = e_ref[...][:, :64]
    t = (jnp.dot(hs_ref[...], ws_ref[...], preferred_element_type=jnp.float32)
         + jnp.dot(hd_ref[...], wd_ref[...], preferred_element_type=jnp.float32)
         + jnp.dot(e, we_ref[...], preferred_element_type=jnp.float32)
         + bs_ref[...])
    m_ref[...] = _mish(t[:, :128])
    en = e + _mish(t[:, 128:])
    enew_ref[...] = jnp.concatenate([en, jnp.zeros_like(en)], axis=1)


def _mid(g1, g2, ep, ws, wd, we, bs):
    blk = 512
    return pl.pallas_call(
        _mid_kernel,
        grid=(E_ALLOC // blk,),
        in_specs=[
            pl.BlockSpec((blk, 128), lambda i: (i, 0)),
            pl.BlockSpec((blk, 128), lambda i: (i, 0)),
            pl.BlockSpec((blk, 128), lambda i: (i, 0)),
            pl.BlockSpec((128, 192), lambda i: (0, 0)),
            pl.BlockSpec((128, 192), lambda i: (0, 0)),
            pl.BlockSpec((64, 192), lambda i: (0, 0)),
            pl.BlockSpec((192,), lambda i: (0,)),
        ],
        out_specs=[
            pl.BlockSpec((blk, 128), lambda i: (i, 0)),
            pl.BlockSpec((blk, 128), lambda i: (i, 0)),
        ],
        out_shape=[
            jax.ShapeDtypeStruct((E_ALLOC, 128), jnp.float32),
            jax.ShapeDtypeStruct((E_ALLOC, 128), jnp.float32),
        ],
    )(g1, g2, ep, ws, wd, we, bs)


def _update_kernel(h_ref, agg_ref, wh_ref, wa_ref, b_ref, g_ref, be_ref,
                   out_ref):
    h = h_ref[...]
    u = (jnp.dot(h, wh_ref[...], preferred_element_type=jnp.float32)
         + jnp.dot(agg_ref[...], wa_ref[...], preferred_element_type=jnp.float32)
         + b_ref[...])
    out_ref[...] = h + _mish(_ln(u, g_ref[...], be_ref[...]))


def _update(hcur, agg, wh, wa, bb, g, be):
    blk = 1000
    return pl.pallas_call(
        _update_kernel,
        grid=(N // blk,),
        in_specs=[
            pl.BlockSpec((blk, 128), lambda i: (i, 0)),
            pl.BlockSpec((blk, 128), lambda i: (i, 0)),
            pl.BlockSpec((128, 128), lambda i: (0, 0)),
            pl.BlockSpec((128, 128), lambda i: (0, 0)),
            pl.BlockSpec((128,), lambda i: (0,)),
            pl.BlockSpec((128,), lambda i: (0,)),
            pl.BlockSpec((128,), lambda i: (0,)),
        ],
        out_specs=pl.BlockSpec((blk, 128), lambda i: (i, 0)),
        out_shape=jax.ShapeDtypeStruct((N, 128), jnp.float32),
    )(hcur, agg, wh, wa, bb, g, be)


def _pool_kernel(h_ref, out_ref):
    out_ref[...] = (jnp.sum(h_ref[...], axis=0, keepdims=True)
                    * (1.0 / (N // B)))[None]


def _pool(node_emb):
    return pl.pallas_call(
        _pool_kernel,
        grid=(B,),
        in_specs=[pl.BlockSpec((N // B, 128), lambda i: (i, 0))],
        out_specs=pl.BlockSpec((1, 1, 128), lambda i: (i, 0, 0)),
        out_shape=jax.ShapeDtypeStruct((B, 1, 128), jnp.float32),
    )(node_emb)


def _head_kernel(ne_ref, ge_ref, w1_ref, b1_ref, g_ref, be_ref, w2_ref, b2_ref,
                 out_ref):
    ne = ne_ref[...]
    ge = jnp.broadcast_to(ge_ref[0], (ne.shape[0], 128))
    u = (jnp.dot(ne, w1_ref[:128], preferred_element_type=jnp.float32)
         + jnp.dot(ge, w1_ref[128:], preferred_element_type=jnp.float32)
         + b1_ref[...])
    sx = _mish(_ln(u, g_ref[...], be_ref[...]))
    out_ref[...] = (jnp.dot(sx, w2_ref[...], preferred_element_type=jnp.float32)
                    + b2_ref[...])


def _head(node_emb, graph_emb, w1, b1, g, be, w2, b2):
    blk = N // B
    w2p = jnp.zeros((256, 128), jnp.float32).at[:, :NUM_CLASSES].set(w2)
    b2p = jnp.zeros((128,), jnp.float32).at[:NUM_CLASSES].set(b2)
    out = pl.pallas_call(
        _head_kernel,
        grid=(B,),
        in_specs=[
            pl.BlockSpec((blk, 128), lambda i: (i, 0)),
            pl.BlockSpec((1, 1, 128), lambda i: (i, 0, 0)),
            pl.BlockSpec((256, 256), lambda i: (0, 0)),
            pl.BlockSpec((256,), lambda i: (0,)),
            pl.BlockSpec((256,), lambda i: (0,)),
            pl.BlockSpec((256,), lambda i: (0,)),
            pl.BlockSpec((256, 128), lambda i: (0, 0)),
            pl.BlockSpec((128,), lambda i: (0,)),
        ],
        out_specs=pl.BlockSpec((blk, 128), lambda i: (i, 0)),
        out_shape=jax.ShapeDtypeStruct((N, 128), jnp.float32),
    )(node_emb, graph_emb, w1, b1, g, be, w2p, b2p)
    return out[:, :NUM_CLASSES]


def _pad_cols(w, total):
    return jnp.concatenate([w, jnp.zeros((w.shape[0], total - w.shape[1]),
                                         w.dtype)], axis=1)


def kernel(node_x, node_grid, edge_x, edge_index, batch_num_nodes, params):
    p = params
    bw1, bb1 = _conv_as_matmul_weights(p['c1_w'], p['c1_b'], p['bn1_g'],
                                       p['bn1_b'])
    bw2, bb2 = _conv_as_matmul_weights(p['c2_w'], p['c2_b'], p['bn2_g'],
                                       p['bn2_b'])
    bw3, bb3 = _conv_as_matmul_weights(p['c3_w'], p['c3_b'], p['bn3_g'],
                                       p['bn3_b'])
    node_feat = _node_enc(node_x, node_grid.reshape(N, 175), p,
                          bw1, bb1, bw2, bb2, bw3, bb3)
    e0 = _edge_enc(edge_x, p)
    src = edge_index[0]
    dst = edge_index[1]
    idr, srr, dsr, lens_flat = _sc_bucket(src, dst)
    lens = lens_flat.reshape(NB, 16)[:, 0]
    lens_pad = ((lens + PCH - 1) // PCH) * PCH
    offs = jnp.concatenate([jnp.zeros((1,), jnp.int32),
                            jnp.cumsum(lens_pad)]).astype(jnp.int32)[:NB]
    srcp, dstp, ep = _sc_permute(idr, srr, dsr, lens, offs, e0)
    hcur = node_feat
    for l in range(NUM_LAYERS):
        mw, mb = p['msg_w%d' % l], p['msg_b%d' % l]
        ew, eb = p['edg_w%d' % l], p['edg_b%d' % l]
        ws = jnp.concatenate([mw[:128], ew[:128]], axis=1)
        wd = jnp.concatenate([mw[128:256], ew[128:256]], axis=1)
        we = jnp.concatenate([mw[256:], ew[256:]], axis=1)
        bs = jnp.concatenate([mb, eb])
        g1, g2 = _sc_gather(hcur, srcp, dstp, lens, offs)
        m, ep = _mid(g1, g2, ep, ws, wd, we, bs)
        agg = _sc_scatter(m, dstp, lens, offs)
        uw, ub = p['upd_w%d' % l], p['upd_b%d' % l]
        hcur = _update(hcur, agg[:N], uw[:128], uw[128:], ub,
                       p['uln_g%d' % l], p['uln_b%d' % l])
    node_emb = hcur
    graph_emb = _pool(node_emb)
    seg = _head(node_emb, graph_emb, p['sh_w1'], p['sh_b1'], p['sh_g'],
                p['sh_be'], p['sh_w2'], p['sh_b2'])
    return seg, node_emb
